# Initial kernel scaffold; baseline (speedup 1.0000x reference)
import jax
import jax.numpy as jnp
from jax.experimental import pallas as pl


def kernel(state, dt, node_attr, edge_attr, glob_attr, params, edge_index, batch, ptr):
    raise NotImplementedError("write your pallas kernel here")


# speedup vs baseline: 2.5858x; 2.5858x over previous
import functools

import jax
import jax.numpy as jnp
from jax import lax
from jax.experimental import pallas as pl
from jax.experimental.pallas import tpu as pltpu
from jax.experimental.pallas import tpu_sc as plsc

N = 10000
E = 320000
B = 8
SD = 8
D = 64
ND = 128
GD = 128

NC = 2
NS = 16
NW = NC * NS
EW = E // NW
CH = 80
NCHUNK = EW // CH
NZ = 624
NTAIL = N - NS * NZ

NBN = 1000
NBE = 4000

_SQRT_HALF = 0.7071067811865476


def _gelu(x):
    return x * 0.5 * (1.0 + lax.erf(x * _SQRT_HALF))


def _dot(a, b):
    return jnp.dot(a, b, preferred_element_type=jnp.float32)


def _mlp2(x, w1, b1, w2, b2):
    h = _gelu(_dot(x, w1[...]) + b1[...])
    return _gelu(_dot(h, w2[...]) + b2[...])


def _onehot_eq(batch_col, nb=B):
    g = lax.broadcasted_iota(jnp.int32, (1, nb), 1)
    return (batch_col == g).astype(jnp.float32)


def _onehot_range(idx_col, lo, hi):
    return ((idx_col >= lo[...]) & (idx_col < hi[...])).astype(jnp.float32)


def _fixed(shape):
    return pl.BlockSpec(shape, lambda i: (0,) * len(shape))


def _k_glob_enc(dt_ref, ga_ref, wd1, bd1, wd2, bd2, wg1, bg1, wg2, bg2, out_ref):
    h = _gelu(dt_ref[...] * wd1[...] + bd1[...])
    de = _gelu(_dot(h, wd2[...]) + bd2[...])
    ge = _mlp2(ga_ref[...], wg1, bg1, wg2, bg2)
    out_ref[...] = jnp.concatenate([de, ge], axis=1)


def _k_encode(st_ref, na_ref, b2_ref, glob_ref,
              ws1, bs1, ws2, bs2, wn1, bn1, wn2, bn2,
              wsrc, wdst, w1g, b1,
              nemb_ref, tab_ref):
    se = _mlp2(st_ref[...], ws1, bs1, ws2, bs2)
    ncoef = _mlp2(na_ref[...], wn1, bn1, wn2, bn2)
    nemb = jnp.concatenate([se, ncoef], axis=1)
    nemb_ref[...] = nemb
    gc = _dot(glob_ref[...], w1g[...])
    oh = _onehot_eq(b2_ref[...])
    a2 = _dot(nemb, wsrc[...]) + _dot(oh, gc) + b1[...]
    bd = _dot(nemb, wdst[...])
    tab_ref[...] = jnp.concatenate([a2, bd], axis=1)


def _k_edge0(gs_ref, gd_ref, ea_ref, src_ref, lo, hi,
             we1, be1, we2, be2, w1e, w2e, b2e,
             out_ref, esum_ref, ecnt_ref):
    i = pl.program_id(0)
    ee = _mlp2(ea_ref[...], we1, be1, we2, be2)
    pre = gs_ref[:, :D] + gd_ref[:, D:] + _dot(ee, w1e[...])
    out = _gelu(_dot(_gelu(pre), w2e[...]) + b2e[...])
    out_ref[...] = jnp.concatenate([out, jnp.zeros_like(out)], axis=1)
    oh = _onehot_range(src_ref[...], lo, hi)
    bs = lax.dot_general(oh, out, (((0,), (0,)), ((), ())),
                         preferred_element_type=jnp.float32)
    bc = lax.dot_general(oh, jnp.ones_like(out), (((0,), (0,)), ((), ())),
                         preferred_element_type=jnp.float32)

    @pl.when(i == 0)
    def _():
        esum_ref[...] = jnp.zeros_like(esum_ref)
        ecnt_ref[...] = jnp.zeros_like(ecnt_ref)

    esum_ref[...] += bs
    ecnt_ref[...] += bc


def _k_edge1(gs_ref, gd_ref, ee_ref, w1e, w2e, b2e, out_ref):
    pre = gs_ref[:, :D] + gd_ref[:, D:] + _dot(ee_ref[:, :D], w1e[...])
    out = _gelu(_dot(_gelu(pre), w2e[...]) + b2e[...])
    out_ref[...] = jnp.concatenate([out, jnp.zeros_like(out)], axis=1)


def _node_new(nemb_ref, aggp_ref, b2_ref, glob_ref, wvn, wva, wvg, bv1, wv2, bv2):
    agg = aggp_ref[0, :, :D] + aggp_ref[1, :, :D]
    gv = _dot(glob_ref[...], wvg[...])
    oh = _onehot_eq(b2_ref[...])
    pre = _dot(nemb_ref[...], wvn[...]) + _dot(agg, wva[...]) + _dot(oh, gv) + bv1[...]
    return _gelu(_dot(_gelu(pre), wv2[...]) + bv2[...]), oh


def _k_node0(nemb_ref, aggp_ref, b2_ref, glob_ref,
             wvn, wva, wvg, bv1, wv2, bv2,
             nnew_ref, nsum_ref, ncnt_ref):
    i = pl.program_id(0)
    nnew, oh = _node_new(nemb_ref, aggp_ref, b2_ref, glob_ref,
                         wvn, wva, wvg, bv1, wv2, bv2)
    nnew_ref[...] = nnew
    bs = lax.dot_general(oh, nnew, (((0,), (0,)), ((), ())),
                         preferred_element_type=jnp.float32)
    bc = lax.dot_general(oh, jnp.ones_like(nnew), (((0,), (0,)), ((), ())),
                         preferred_element_type=jnp.float32)

    @pl.when(i == 0)
    def _():
        nsum_ref[...] = jnp.zeros_like(nsum_ref)
        ncnt_ref[...] = jnp.zeros_like(ncnt_ref)

    nsum_ref[...] += bs
    ncnt_ref[...] += bc


def _k_node1(nemb_ref, aggp_ref, b2_ref, glob_ref, st_ref,
             wvn, wva, wvg, bv1, wv2, bv2, wd1, bd1, wd2, bd2,
             out_ref):
    nnew, _ = _node_new(nemb_ref, aggp_ref, b2_ref, glob_ref,
                        wvn, wva, wvg, bv1, wv2, bv2)
    h = _gelu(_dot(nnew, wd1[...]) + bd1[...])
    out_ref[...] = st_ref[...] + _dot(h, wd2[...]) + bd2[...]


def _k_globpre(nemb_ref, b2_ref, glob_ref, nsum, ncnt, esum, ecnt,
               wgn, wge, wgg, bg1, wg2, bg2, wsrc, wdst, w1g, b1,
               tab_ref, glob1_ref):
    nmean = nsum[...] / jnp.maximum(ncnt[...], 1.0)
    emean = esum[...] / jnp.maximum(ecnt[...], 1.0)
    gpre = _dot(nmean, wgn[...]) + _dot(emean, wge[...]) + _dot(glob_ref[...], wgg[...]) + bg1[...]
    glob1 = _gelu(_dot(_gelu(gpre), wg2[...]) + bg2[...])
    glob1_ref[...] = glob1
    gc = _dot(glob1, w1g[...])
    oh = _onehot_eq(b2_ref[...])
    a2 = _dot(nemb_ref[...], wsrc[...]) + _dot(oh, gc) + b1[...]
    bd = _dot(nemb_ref[...], wdst[...])
    tab_ref[...] = jnp.concatenate([a2, bd], axis=1)


_TC_PARAMS = pltpu.CompilerParams(dimension_semantics=("arbitrary",))


def _glob_enc(dt, glob_attr, wd1, bd1, wd2, bd2, wg1, bg1, wg2, bg2):
    return pl.pallas_call(
        _k_glob_enc,
        out_shape=jax.ShapeDtypeStruct((B, GD), jnp.float32),
    )(dt, glob_attr, wd1, bd1, wd2, bd2, wg1, bg1, wg2, bg2)


def _encode(state, node_attr, batch2, glob0, *ws):
    grid = (N // NBN,)
    return pl.pallas_call(
        _k_encode,
        grid=grid,
        in_specs=[
            pl.BlockSpec((NBN, SD), lambda i: (i, 0)),
            pl.BlockSpec((NBN, SD), lambda i: (i, 0)),
            pl.BlockSpec((NBN, 1), lambda i: (i, 0)),
            _fixed((B, GD)),
        ] + [_fixed(w.shape) for w in ws],
        out_specs=[
            pl.BlockSpec((NBN, ND), lambda i: (i, 0)),
            pl.BlockSpec((NBN, 2 * D), lambda i: (i, 0)),
        ],
        out_shape=[
            jax.ShapeDtypeStruct((N, ND), jnp.float32),
            jax.ShapeDtypeStruct((N, 2 * D), jnp.float32),
        ],
        compiler_params=_TC_PARAMS,
    )(state, node_attr, batch2, glob0, *ws)


def _edge0(gs, gd, edge_attr, src2, lo, hi, *ws):
    grid = (E // NBE,)
    return pl.pallas_call(
        _k_edge0,
        grid=grid,
        in_specs=[
            pl.BlockSpec((NBE, 2 * D), lambda i: (i, 0)),
            pl.BlockSpec((NBE, 2 * D), lambda i: (i, 0)),
            pl.BlockSpec((NBE, 4), lambda i: (i, 0)),
            pl.BlockSpec((NBE, 1), lambda i: (i, 0)),
            _fixed((1, B)),
            _fixed((1, B)),
        ] + [_fixed(w.shape) for w in ws],
        out_specs=[
            pl.BlockSpec((NBE, 2 * D), lambda i: (i, 0)),
            _fixed((B, D)),
            _fixed((B, D)),
        ],
        out_shape=[
            jax.ShapeDtypeStruct((E, 2 * D), jnp.float32),
            jax.ShapeDtypeStruct((B, D), jnp.float32),
            jax.ShapeDtypeStruct((B, D), jnp.float32),
        ],
        compiler_params=_TC_PARAMS,
    )(gs, gd, edge_attr, src2, lo, hi, *ws)


def _edge1(gs, gd, ee, *ws):
    grid = (E // NBE,)
    return pl.pallas_call(
        _k_edge1,
        grid=grid,
        in_specs=[
            pl.BlockSpec((NBE, 2 * D), lambda i: (i, 0)),
            pl.BlockSpec((NBE, 2 * D), lambda i: (i, 0)),
            pl.BlockSpec((NBE, 2 * D), lambda i: (i, 0)),
        ] + [_fixed(w.shape) for w in ws],
        out_specs=pl.BlockSpec((NBE, 2 * D), lambda i: (i, 0)),
        out_shape=jax.ShapeDtypeStruct((E, 2 * D), jnp.float32),
        compiler_params=_TC_PARAMS,
    )(gs, gd, ee, *ws)


def _node0(nemb, aggp, batch2, glob0, *ws):
    grid = (N // NBN,)
    return pl.pallas_call(
        _k_node0,
        grid=grid,
        in_specs=[
            pl.BlockSpec((NBN, ND), lambda i: (i, 0)),
            pl.BlockSpec((NC, NBN, 2 * D), lambda i: (0, i, 0)),
            pl.BlockSpec((NBN, 1), lambda i: (i, 0)),
            _fixed((B, GD)),
        ] + [_fixed(w.shape) for w in ws],
        out_specs=[
            pl.BlockSpec((NBN, ND), lambda i: (i, 0)),
            _fixed((B, ND)),
            _fixed((B, ND)),
        ],
        out_shape=[
            jax.ShapeDtypeStruct((N, ND), jnp.float32),
            jax.ShapeDtypeStruct((B, ND), jnp.float32),
            jax.ShapeDtypeStruct((B, ND), jnp.float32),
        ],
        compiler_params=_TC_PARAMS,
    )(nemb, aggp, batch2, glob0, *ws)


def _node1(nemb, aggp, batch2, glob1, state, *ws):
    grid = (N // NBN,)
    return pl.pallas_call(
        _k_node1,
        grid=grid,
        in_specs=[
            pl.BlockSpec((NBN, ND), lambda i: (i, 0)),
            pl.BlockSpec((NC, NBN, 2 * D), lambda i: (0, i, 0)),
            pl.BlockSpec((NBN, 1), lambda i: (i, 0)),
            _fixed((B, GD)),
            pl.BlockSpec((NBN, SD), lambda i: (i, 0)),
        ] + [_fixed(w.shape) for w in ws],
        out_specs=pl.BlockSpec((NBN, SD), lambda i: (i, 0)),
        out_shape=jax.ShapeDtypeStruct((N, SD), jnp.float32),
        compiler_params=_TC_PARAMS,
    )(nemb, aggp, batch2, glob1, state, *ws)


def _globpre(nemb1, batch2, glob0, nsum, ncnt, esum, ecnt, *ws):
    grid = (N // NBN,)
    return pl.pallas_call(
        _k_globpre,
        grid=grid,
        in_specs=[
            pl.BlockSpec((NBN, ND), lambda i: (i, 0)),
            pl.BlockSpec((NBN, 1), lambda i: (i, 0)),
            _fixed((B, GD)),
            _fixed((B, ND)),
            _fixed((B, ND)),
            _fixed((B, D)),
            _fixed((B, D)),
        ] + [_fixed(w.shape) for w in ws],
        out_specs=[
            pl.BlockSpec((NBN, 2 * D), lambda i: (i, 0)),
            _fixed((B, GD)),
        ],
        out_shape=[
            jax.ShapeDtypeStruct((N, 2 * D), jnp.float32),
            jax.ShapeDtypeStruct((B, GD), jnp.float32),
        ],
        compiler_params=_TC_PARAMS,
    )(nemb1, batch2, glob0, nsum, ncnt, esum, ecnt, *ws)


def _sc_gather_body(tab_hbm, src_hbm, dst_hbm, ga_hbm, gb_hbm,
                    idx_a, idx_b, rows_a, rows_b, sem_a, sem_b):
    c = lax.axis_index("c")
    s = lax.axis_index("s")
    wid = s * NC + c
    pltpu.sync_copy(src_hbm.at[wid], idx_a)
    pltpu.sync_copy(dst_hbm.at[wid], idx_b)

    def body(j, carry):
        cpa = pltpu.async_copy(tab_hbm.at[idx_a.at[j]], rows_a, sem_a)
        cpb = pltpu.async_copy(tab_hbm.at[idx_b.at[j]], rows_b, sem_b)
        cpa.wait()
        cpb.wait()
        base = wid * EW + j * CH
        pltpu.sync_copy(rows_a, ga_hbm.at[pl.ds(base, CH)])
        pltpu.sync_copy(rows_b, gb_hbm.at[pl.ds(base, CH)])
        return carry

    lax.fori_loop(0, NCHUNK, body, 0)


def _sc_scatter_body(e_hbm, dst_hbm, zero_hbm, out_hbm, idx_v, rows_v, acc):
    c = lax.axis_index("c")
    s = lax.axis_index("s")
    wid = s * NC + c
    pltpu.sync_copy(zero_hbm.at[pl.ds(0, NZ)], acc.at[pl.ds(s * NZ, NZ)])

    @pl.when(s == NS - 1)
    def _():
        pltpu.sync_copy(zero_hbm.at[pl.ds(0, NTAIL)], acc.at[pl.ds(NS * NZ, NTAIL)])

    plsc.subcore_barrier()
    pltpu.sync_copy(dst_hbm.at[wid], idx_v)

    def body(j, carry):
        base = wid * EW + j * CH
        pltpu.sync_copy(e_hbm.at[pl.ds(base, CH)], rows_v)
        pltpu.sync_copy(rows_v, acc.at[idx_v.at[j]], add=True)
        return carry

    lax.fori_loop(0, NCHUNK, body, 0)
    plsc.subcore_barrier()
    pltpu.sync_copy(acc.at[pl.ds(s * NZ, NZ)], out_hbm.at[c, pl.ds(s * NZ, NZ)])

    @pl.when(s == NS - 1)
    def _():
        pltpu.sync_copy(acc.at[pl.ds(NS * NZ, NTAIL)],
                        out_hbm.at[c, pl.ds(NS * NZ, NTAIL)])


@functools.cache
def _sc_kernels():
    mesh = plsc.VectorSubcoreMesh(core_axis_name="c", subcore_axis_name="s",
                                  num_cores=NC, num_subcores=NS)
    gather = pl.kernel(
        _sc_gather_body,
        out_type=(jax.ShapeDtypeStruct((E, 2 * D), jnp.float32),
                  jax.ShapeDtypeStruct((E, 2 * D), jnp.float32)),
        mesh=mesh,
        scratch_types=[
            pltpu.VMEM((NCHUNK, CH), jnp.int32),
            pltpu.VMEM((NCHUNK, CH), jnp.int32),
            pltpu.VMEM((CH, 2 * D), jnp.float32),
            pltpu.VMEM((CH, 2 * D), jnp.float32),
            pltpu.SemaphoreType.DMA,
            pltpu.SemaphoreType.DMA,
        ],
    )
    scatter = pl.kernel(
        _sc_scatter_body,
        out_type=jax.ShapeDtypeStruct((NC, N, 2 * D), jnp.float32),
        mesh=mesh,
        scratch_types=[
            pltpu.VMEM((NCHUNK, CH), jnp.int32),
            pltpu.VMEM((CH, 2 * D), jnp.float32),
            pltpu.VMEM_SHARED((N, 2 * D), jnp.float32),
        ],
    )
    return gather, scatter


def _sc_gather(tab, src3, dst3):
    return tab[src3.reshape(-1)], tab[dst3.reshape(-1)]


def _sc_scatter(edge_emb, dst3, zeros):
    return _sc_kernels()[1](edge_emb, dst3, zeros)


def _lin_w(p):
    return p["W"], p["b"].reshape(1, -1)


def kernel(state, dt, node_attr, edge_attr, glob_attr, params, edge_index, batch, ptr):
    src = edge_index[0]
    dst = edge_index[1]
    src3 = src.reshape(NW, NCHUNK, CH)
    dst3 = dst.reshape(NW, NCHUNK, CH)
    src2 = src.reshape(E, 1)
    batch2 = batch.reshape(N, 1)
    lo = ptr[:B].reshape(1, B)
    hi = ptr[1:B + 1].reshape(1, B)
    zeros = jnp.zeros((NZ, 2 * D), jnp.float32)

    ws1, bs1 = _lin_w(params["state_enc"]["l1"])
    ws2, bs2 = _lin_w(params["state_enc"]["l2"])
    wn1, bn1 = _lin_w(params["node_enc"]["l1"])
    wn2, bn2 = _lin_w(params["node_enc"]["l2"])
    wd1_, bd1_ = _lin_w(params["dt_enc"]["l1"])
    wd2_, bd2_ = _lin_w(params["dt_enc"]["l2"])
    wg1_, bg1_ = _lin_w(params["glob_enc"]["l1"])
    wg2_, bg2_ = _lin_w(params["glob_enc"]["l2"])
    we1, be1 = _lin_w(params["edge_enc"]["l1"])
    we2, be2 = _lin_w(params["edge_enc"]["l2"])
    wdec1, bdec1 = _lin_w(params["dec"]["l1"])
    wdec2, bdec2 = _lin_w(params["dec"]["l2"])

    layers = []
    for layer in params["gn"]:
        w1, b1 = _lin_w(layer["phi_e"]["l1"])
        w2e, b2e = _lin_w(layer["phi_e"]["l2"])
        wv1, bv1 = _lin_w(layer["phi_v"]["l1"])
        wv2, bv2 = _lin_w(layer["phi_v"]["l2"])
        wg1, bg1 = _lin_w(layer["phi_g"]["l1"])
        wg2, bg2 = _lin_w(layer["phi_g"]["l2"])
        layers.append(dict(
            wsrc=w1[:ND], wdst=w1[ND:2 * ND], w1e=w1[2 * ND:2 * ND + D],
            w1g=w1[2 * ND + D:], b1=b1, w2e=w2e, b2e=b2e,
            wvn=wv1[:ND], wva=wv1[ND:ND + D], wvg=wv1[ND + D:], bv1=bv1,
            wv2=wv2, bv2=bv2,
            wgn=wg1[:ND], wge=wg1[ND:ND + D], wgg=wg1[ND + D:], bg1=bg1,
            wg2=wg2, bg2=bg2,
        ))
    l0, l1 = layers

    glob0 = _glob_enc(dt, glob_attr, wd1_, bd1_, wd2_, bd2_, wg1_, bg1_, wg2_, bg2_)
    nemb, tab0 = _encode(state, node_attr, batch2, glob0,
                         ws1, bs1, ws2, bs2, wn1, bn1, wn2, bn2,
                         l0["wsrc"], l0["wdst"], l0["w1g"], l0["b1"])
    gs0, gd0 = _sc_gather(tab0, src3, dst3)
    edge1, esum, ecnt = _edge0(gs0, gd0, edge_attr, src2, lo, hi,
                               we1, be1, we2, be2, l0["w1e"], l0["w2e"], l0["b2e"])
    aggp0 = _sc_scatter(edge1, dst3, zeros)
    nemb1, nsum, ncnt = _node0(nemb, aggp0, batch2, glob0,
                               l0["wvn"], l0["wva"], l0["wvg"], l0["bv1"],
                               l0["wv2"], l0["bv2"])
    tab1, glob1 = _globpre(nemb1, batch2, glob0, nsum, ncnt, esum, ecnt,
                           l0["wgn"], l0["wge"], l0["wgg"], l0["bg1"],
                           l0["wg2"], l0["bg2"],
                           l1["wsrc"], l1["wdst"], l1["w1g"], l1["b1"])
    gs1, gd1 = _sc_gather(tab1, src3, dst3)
    edge2 = _edge1(gs1, gd1, edge1, l1["w1e"], l1["w2e"], l1["b2e"])
    aggp1 = _sc_scatter(edge2, dst3, zeros)
    out = _node1(nemb1, aggp1, batch2, glob1, state,
                 l1["wvn"], l1["wva"], l1["wvg"], l1["bv1"],
                 l1["wv2"], l1["bv2"], wdec1, bdec1, wdec2, bdec2)
    return out


# speedup vs baseline: 4.6692x; 1.8057x over previous
import functools

import jax
import jax.numpy as jnp
from jax import lax
from jax.experimental import pallas as pl
from jax.experimental.pallas import tpu as pltpu
from jax.experimental.pallas import tpu_sc as plsc

N = 10000
E = 320000
B = 8
SD = 8
EDGE_A = 4
D = 64
ND = 128
GD = 128

NH = 2
EH = E // NH
NC = 2
NS = 16
NW = NC * NS
EW = EH // NW
CH = 40
NCHUNK = EW // CH
NZ = 624
NTAIL = N - NS * NZ

NBN = 1000
NBE = 4000

_SQRT_HALF = 0.7071067811865476


def _gelu(x):
    return x * 0.5 * (1.0 + lax.erf(x * _SQRT_HALF))


def _dot(a, b):
    return jnp.dot(a, b, preferred_element_type=jnp.float32)


def _mlp2(x, w1, b1, w2, b2):
    h = _gelu(_dot(x, w1[...]) + b1[...])
    return _gelu(_dot(h, w2[...]) + b2[...])


def _onehot_eq(batch_col, nb=B):
    g = lax.broadcasted_iota(jnp.int32, (1, nb), 1)
    return (batch_col == g).astype(jnp.float32)


def _onehot_range(idx_col, lo, hi):
    return ((idx_col >= lo[...]) & (idx_col < hi[...])).astype(jnp.float32)


def _fixed(shape):
    return pl.BlockSpec(shape, lambda i: (0,) * len(shape))


def _k_glob_enc(dt_ref, ga_ref, wd1, bd1, wd2, bd2, wg1, bg1, wg2, bg2, out_ref):
    h = _gelu(dt_ref[...] * wd1[...] + bd1[...])
    de = _gelu(_dot(h, wd2[...]) + bd2[...])
    ge = _mlp2(ga_ref[...], wg1, bg1, wg2, bg2)
    out_ref[...] = jnp.concatenate([de, ge], axis=1)


def _k_encode(st_ref, na_ref, b2_ref, glob_ref,
              ws1, bs1, ws2, bs2, wn1, bn1, wn2, bn2,
              wsrc, wdst, w1g, b1,
              nemb_ref, tab_ref):
    se = _mlp2(st_ref[...], ws1, bs1, ws2, bs2)
    ncoef = _mlp2(na_ref[...], wn1, bn1, wn2, bn2)
    nemb = jnp.concatenate([se, ncoef], axis=1)
    nemb_ref[...] = nemb
    gc = _dot(glob_ref[...], w1g[...])
    oh = _onehot_eq(b2_ref[...])
    a2 = _dot(nemb, wsrc[...]) + _dot(oh, gc) + b1[...]
    bd = _dot(nemb, wdst[...])
    tab_ref[...] = jnp.concatenate([a2, bd], axis=1)


def _k_edge0(gs_ref, gd_ref, ea_ref, src_ref, lo, hi,
             we1, be1, we2, be2, w1e, w2e, b2e,
             out_ref, esum_ref, ecnt_ref):
    i = pl.program_id(0)
    ee = _mlp2(ea_ref[...], we1, be1, we2, be2)
    pre = gs_ref[:, :D] + gd_ref[:, D:] + _dot(ee, w1e[...])
    out = _gelu(_dot(_gelu(pre), w2e[...]) + b2e[...])
    out_ref[...] = jnp.concatenate([out, jnp.zeros_like(out)], axis=1)
    oh = _onehot_range(src_ref[...], lo, hi)
    bs = lax.dot_general(oh, out, (((0,), (0,)), ((), ())),
                         preferred_element_type=jnp.float32)
    bc = lax.dot_general(oh, jnp.ones_like(out), (((0,), (0,)), ((), ())),
                         preferred_element_type=jnp.float32)

    @pl.when(i == 0)
    def _():
        esum_ref[...] = jnp.zeros_like(esum_ref)
        ecnt_ref[...] = jnp.zeros_like(ecnt_ref)

    esum_ref[...] += bs
    ecnt_ref[...] += bc


def _k_edge1(gs_ref, gd_ref, ee_ref, w1e, w2e, b2e, out_ref):
    pre = gs_ref[:, :D] + gd_ref[:, D:] + _dot(ee_ref[:, :D], w1e[...])
    out = _gelu(_dot(_gelu(pre), w2e[...]) + b2e[...])
    out_ref[...] = jnp.concatenate([out, jnp.zeros_like(out)], axis=1)


def _node_new(nemb_ref, agg1_ref, agg2_ref, b2_ref, glob_ref,
              wvn, wva, wvg, bv1, wv2, bv2):
    agg = (agg1_ref[0, :, :D] + agg1_ref[1, :, :D]
           + agg2_ref[0, :, :D] + agg2_ref[1, :, :D])
    gv = _dot(glob_ref[...], wvg[...])
    oh = _onehot_eq(b2_ref[...])
    pre = _dot(nemb_ref[...], wvn[...]) + _dot(agg, wva[...]) + _dot(oh, gv) + bv1[...]
    return _gelu(_dot(_gelu(pre), wv2[...]) + bv2[...]), oh


def _k_node0(nemb_ref, agg1_ref, agg2_ref, b2_ref, glob_ref,
             wvn, wva, wvg, bv1, wv2, bv2,
             nnew_ref, nsum_ref, ncnt_ref):
    i = pl.program_id(0)
    nnew, oh = _node_new(nemb_ref, agg1_ref, agg2_ref, b2_ref, glob_ref,
                         wvn, wva, wvg, bv1, wv2, bv2)
    nnew_ref[...] = nnew
    bs = lax.dot_general(oh, nnew, (((0,), (0,)), ((), ())),
                         preferred_element_type=jnp.float32)
    bc = lax.dot_general(oh, jnp.ones_like(nnew), (((0,), (0,)), ((), ())),
                         preferred_element_type=jnp.float32)

    @pl.when(i == 0)
    def _():
        nsum_ref[...] = jnp.zeros_like(nsum_ref)
        ncnt_ref[...] = jnp.zeros_like(ncnt_ref)

    nsum_ref[...] += bs
    ncnt_ref[...] += bc


def _k_node1(nemb_ref, agg1_ref, agg2_ref, b2_ref, glob_ref, st_ref,
             wvn, wva, wvg, bv1, wv2, bv2, wd1, bd1, wd2, bd2,
             out_ref):
    nnew, _ = _node_new(nemb_ref, agg1_ref, agg2_ref, b2_ref, glob_ref,
                        wvn, wva, wvg, bv1, wv2, bv2)
    h = _gelu(_dot(nnew, wd1[...]) + bd1[...])
    out_ref[...] = st_ref[...] + _dot(h, wd2[...]) + bd2[...]


def _k_globpre(nemb_ref, b2_ref, glob_ref, nsum, ncnt, esum1, ecnt1, esum2, ecnt2,
               wgn, wge, wgg, bg1, wg2, bg2, wsrc, wdst, w1g, b1,
               tab_ref, glob1_ref):
    nmean = nsum[...] / jnp.maximum(ncnt[...], 1.0)
    emean = (esum1[...] + esum2[...]) / jnp.maximum(ecnt1[...] + ecnt2[...], 1.0)
    gpre = _dot(nmean, wgn[...]) + _dot(emean, wge[...]) + _dot(glob_ref[...], wgg[...]) + bg1[...]
    glob1 = _gelu(_dot(_gelu(gpre), wg2[...]) + bg2[...])
    glob1_ref[...] = glob1
    gc = _dot(glob1, w1g[...])
    oh = _onehot_eq(b2_ref[...])
    a2 = _dot(nemb_ref[...], wsrc[...]) + _dot(oh, gc) + b1[...]
    bd = _dot(nemb_ref[...], wdst[...])
    tab_ref[...] = jnp.concatenate([a2, bd], axis=1)


_TC_PARAMS = pltpu.CompilerParams(dimension_semantics=("arbitrary",))


def _glob_enc(dt, glob_attr, wd1, bd1, wd2, bd2, wg1, bg1, wg2, bg2):
    return pl.pallas_call(
        _k_glob_enc,
        out_shape=jax.ShapeDtypeStruct((B, GD), jnp.float32),
    )(dt, glob_attr, wd1, bd1, wd2, bd2, wg1, bg1, wg2, bg2)


def _encode(state, node_attr, batch2, glob0, *ws):
    grid = (N // NBN,)
    return pl.pallas_call(
        _k_encode,
        grid=grid,
        in_specs=[
            pl.BlockSpec((NBN, SD), lambda i: (i, 0)),
            pl.BlockSpec((NBN, SD), lambda i: (i, 0)),
            pl.BlockSpec((NBN, 1), lambda i: (i, 0)),
            _fixed((B, GD)),
        ] + [_fixed(w.shape) for w in ws],
        out_specs=[
            pl.BlockSpec((NBN, ND), lambda i: (i, 0)),
            pl.BlockSpec((NBN, 2 * D), lambda i: (i, 0)),
        ],
        out_shape=[
            jax.ShapeDtypeStruct((N, ND), jnp.float32),
            jax.ShapeDtypeStruct((N, 2 * D), jnp.float32),
        ],
        compiler_params=_TC_PARAMS,
    )(state, node_attr, batch2, glob0, *ws)


def _edge0(gs, gd, edge_attr, src2, lo, hi, *ws):
    grid = (EH // NBE,)
    return pl.pallas_call(
        _k_edge0,
        grid=grid,
        in_specs=[
            pl.BlockSpec((NBE, 2 * D), lambda i: (i, 0)),
            pl.BlockSpec((NBE, 2 * D), lambda i: (i, 0)),
            pl.BlockSpec((NBE, 4), lambda i: (i, 0)),
            pl.BlockSpec((NBE, 1), lambda i: (i, 0)),
            _fixed((1, B)),
            _fixed((1, B)),
        ] + [_fixed(w.shape) for w in ws],
        out_specs=[
            pl.BlockSpec((NBE, 2 * D), lambda i: (i, 0)),
            _fixed((B, D)),
            _fixed((B, D)),
        ],
        out_shape=[
            jax.ShapeDtypeStruct((EH, 2 * D), jnp.float32),
            jax.ShapeDtypeStruct((B, D), jnp.float32),
            jax.ShapeDtypeStruct((B, D), jnp.float32),
        ],
        compiler_params=_TC_PARAMS,
    )(gs, gd, edge_attr, src2, lo, hi, *ws)


def _edge1(gs, gd, ee, *ws):
    grid = (EH // NBE,)
    return pl.pallas_call(
        _k_edge1,
        grid=grid,
        in_specs=[
            pl.BlockSpec((NBE, 2 * D), lambda i: (i, 0)),
            pl.BlockSpec((NBE, 2 * D), lambda i: (i, 0)),
            pl.BlockSpec((NBE, 2 * D), lambda i: (i, 0)),
        ] + [_fixed(w.shape) for w in ws],
        out_specs=pl.BlockSpec((NBE, 2 * D), lambda i: (i, 0)),
        out_shape=jax.ShapeDtypeStruct((EH, 2 * D), jnp.float32),
        compiler_params=_TC_PARAMS,
    )(gs, gd, ee, *ws)


def _node0(nemb, agg1, agg2, batch2, glob0, *ws):
    grid = (N // NBN,)
    return pl.pallas_call(
        _k_node0,
        grid=grid,
        in_specs=[
            pl.BlockSpec((NBN, ND), lambda i: (i, 0)),
            pl.BlockSpec((NC, NBN, 2 * D), lambda i: (0, i, 0)),
            pl.BlockSpec((NC, NBN, 2 * D), lambda i: (0, i, 0)),
            pl.BlockSpec((NBN, 1), lambda i: (i, 0)),
            _fixed((B, GD)),
        ] + [_fixed(w.shape) for w in ws],
        out_specs=[
            pl.BlockSpec((NBN, ND), lambda i: (i, 0)),
            _fixed((B, ND)),
            _fixed((B, ND)),
        ],
        out_shape=[
            jax.ShapeDtypeStruct((N, ND), jnp.float32),
            jax.ShapeDtypeStruct((B, ND), jnp.float32),
            jax.ShapeDtypeStruct((B, ND), jnp.float32),
        ],
        compiler_params=_TC_PARAMS,
    )(nemb, agg1, agg2, batch2, glob0, *ws)


def _node1(nemb, agg1, agg2, batch2, glob1, state, *ws):
    grid = (N // NBN,)
    return pl.pallas_call(
        _k_node1,
        grid=grid,
        in_specs=[
            pl.BlockSpec((NBN, ND), lambda i: (i, 0)),
            pl.BlockSpec((NC, NBN, 2 * D), lambda i: (0, i, 0)),
            pl.BlockSpec((NC, NBN, 2 * D), lambda i: (0, i, 0)),
            pl.BlockSpec((NBN, 1), lambda i: (i, 0)),
            _fixed((B, GD)),
            pl.BlockSpec((NBN, SD), lambda i: (i, 0)),
        ] + [_fixed(w.shape) for w in ws],
        out_specs=pl.BlockSpec((NBN, SD), lambda i: (i, 0)),
        out_shape=jax.ShapeDtypeStruct((N, SD), jnp.float32),
        compiler_params=_TC_PARAMS,
    )(nemb, agg1, agg2, batch2, glob1, state, *ws)


def _globpre(nemb1, batch2, glob0, nsum, ncnt, esum1, ecnt1, esum2, ecnt2, *ws):
    grid = (N // NBN,)
    return pl.pallas_call(
        _k_globpre,
        grid=grid,
        in_specs=[
            pl.BlockSpec((NBN, ND), lambda i: (i, 0)),
            pl.BlockSpec((NBN, 1), lambda i: (i, 0)),
            _fixed((B, GD)),
            _fixed((B, ND)),
            _fixed((B, ND)),
            _fixed((B, D)),
            _fixed((B, D)),
            _fixed((B, D)),
            _fixed((B, D)),
        ] + [_fixed(w.shape) for w in ws],
        out_specs=[
            pl.BlockSpec((NBN, 2 * D), lambda i: (i, 0)),
            _fixed((B, GD)),
        ],
        out_shape=[
            jax.ShapeDtypeStruct((N, 2 * D), jnp.float32),
            jax.ShapeDtypeStruct((B, GD), jnp.float32),
        ],
        compiler_params=_TC_PARAMS,
    )(nemb1, batch2, glob0, nsum, ncnt, esum1, ecnt1, esum2, ecnt2, *ws)


def _sc_gather_body(tab_hbm, src_hbm, dst_hbm, ga_hbm, gb_hbm,
                    idx_a, idx_b, rows_a, rows_b, sem_a, sem_b):
    c = lax.axis_index("c")
    s = lax.axis_index("s")
    wid = s * NC + c
    pltpu.sync_copy(src_hbm.at[wid], idx_a)
    pltpu.sync_copy(dst_hbm.at[wid], idx_b)

    def body(j, carry):
        cpa = pltpu.async_copy(tab_hbm.at[idx_a.at[j]], rows_a, sem_a)
        cpb = pltpu.async_copy(tab_hbm.at[idx_b.at[j]], rows_b, sem_b)
        cpa.wait()
        cpb.wait()
        base = wid * EW + j * CH
        pltpu.sync_copy(rows_a, ga_hbm.at[pl.ds(base, CH)])
        pltpu.sync_copy(rows_b, gb_hbm.at[pl.ds(base, CH)])
        return carry

    lax.fori_loop(0, NCHUNK, body, 0)


def _sc_scatter_body(e_hbm, dst_hbm, zero_hbm, out_hbm, idx_v, rows_v, acc):
    c = lax.axis_index("c")
    s = lax.axis_index("s")
    wid = s * NC + c
    pltpu.sync_copy(zero_hbm.at[pl.ds(0, NZ)], acc.at[pl.ds(s * NZ, NZ)])

    @pl.when(s == NS - 1)
    def _():
        pltpu.sync_copy(zero_hbm.at[pl.ds(0, NTAIL)], acc.at[pl.ds(NS * NZ, NTAIL)])

    plsc.subcore_barrier()
    pltpu.sync_copy(dst_hbm.at[wid], idx_v)

    def body(j, carry):
        base = wid * EW + j * CH
        pltpu.sync_copy(e_hbm.at[pl.ds(base, CH)], rows_v)
        pltpu.sync_copy(rows_v, acc.at[idx_v.at[j]], add=True)
        return carry

    lax.fori_loop(0, NCHUNK, body, 0)
    plsc.subcore_barrier()
    pltpu.sync_copy(acc.at[pl.ds(s * NZ, NZ)], out_hbm.at[c, pl.ds(s * NZ, NZ)])

    @pl.when(s == NS - 1)
    def _():
        pltpu.sync_copy(acc.at[pl.ds(NS * NZ, NTAIL)],
                        out_hbm.at[c, pl.ds(NS * NZ, NTAIL)])


@functools.cache
def _sc_kernels():
    mesh = plsc.VectorSubcoreMesh(core_axis_name="c", subcore_axis_name="s",
                                  num_cores=NC, num_subcores=NS)
    gather = pl.kernel(
        _sc_gather_body,
        out_type=(jax.ShapeDtypeStruct((EH, 2 * D), jnp.float32),
                  jax.ShapeDtypeStruct((EH, 2 * D), jnp.float32)),
        mesh=mesh,
        scratch_types=[
            pltpu.VMEM((NCHUNK, CH), jnp.int32),
            pltpu.VMEM((NCHUNK, CH), jnp.int32),
            pltpu.VMEM((CH, 2 * D), jnp.float32),
            pltpu.VMEM((CH, 2 * D), jnp.float32),
            pltpu.SemaphoreType.DMA,
            pltpu.SemaphoreType.DMA,
        ],
    )
    scatter = pl.kernel(
        _sc_scatter_body,
        out_type=jax.ShapeDtypeStruct((NC, N, 2 * D), jnp.float32),
        mesh=mesh,
        scratch_types=[
            pltpu.VMEM((NCHUNK, CH), jnp.int32),
            pltpu.VMEM((CH, 2 * D), jnp.float32),
            pltpu.VMEM_SHARED((N, 2 * D), jnp.float32),
        ],
    )
    return gather, scatter


def _sc_gather(tab, src3, dst3):
    return _sc_kernels()[0](tab, src3, dst3)


def _sc_scatter(edge_emb, dst3, zeros):
    return _sc_kernels()[1](edge_emb, dst3, zeros)


def _lin_w(p):
    return p["W"], p["b"].reshape(1, -1)


def kernel(state, dt, node_attr, edge_attr, glob_attr, params, edge_index, batch, ptr):
    src = edge_index[0]
    dst = edge_index[1]
    src3 = src.reshape(NH, NW, NCHUNK, CH)
    dst3 = dst.reshape(NH, NW, NCHUNK, CH)
    src2 = src.reshape(NH, EH, 1)
    ea_h = edge_attr.reshape(NH, EH, EDGE_A)
    batch2 = batch.reshape(N, 1)
    lo = ptr[:B].reshape(1, B)
    hi = ptr[1:B + 1].reshape(1, B)
    zeros = jnp.zeros((NZ, 2 * D), jnp.float32)

    ws1, bs1 = _lin_w(params["state_enc"]["l1"])
    ws2, bs2 = _lin_w(params["state_enc"]["l2"])
    wn1, bn1 = _lin_w(params["node_enc"]["l1"])
    wn2, bn2 = _lin_w(params["node_enc"]["l2"])
    wd1_, bd1_ = _lin_w(params["dt_enc"]["l1"])
    wd2_, bd2_ = _lin_w(params["dt_enc"]["l2"])
    wg1_, bg1_ = _lin_w(params["glob_enc"]["l1"])
    wg2_, bg2_ = _lin_w(params["glob_enc"]["l2"])
    we1, be1 = _lin_w(params["edge_enc"]["l1"])
    we2, be2 = _lin_w(params["edge_enc"]["l2"])
    wdec1, bdec1 = _lin_w(params["dec"]["l1"])
    wdec2, bdec2 = _lin_w(params["dec"]["l2"])

    layers = []
    for layer in params["gn"]:
        w1, b1 = _lin_w(layer["phi_e"]["l1"])
        w2e, b2e = _lin_w(layer["phi_e"]["l2"])
        wv1, bv1 = _lin_w(layer["phi_v"]["l1"])
        wv2, bv2 = _lin_w(layer["phi_v"]["l2"])
        wg1, bg1 = _lin_w(layer["phi_g"]["l1"])
        wg2, bg2 = _lin_w(layer["phi_g"]["l2"])
        layers.append(dict(
            wsrc=w1[:ND], wdst=w1[ND:2 * ND], w1e=w1[2 * ND:2 * ND + D],
            w1g=w1[2 * ND + D:], b1=b1, w2e=w2e, b2e=b2e,
            wvn=wv1[:ND], wva=wv1[ND:ND + D], wvg=wv1[ND + D:], bv1=bv1,
            wv2=wv2, bv2=bv2,
            wgn=wg1[:ND], wge=wg1[ND:ND + D], wgg=wg1[ND + D:], bg1=bg1,
            wg2=wg2, bg2=bg2,
        ))
    l0, l1 = layers

    glob0 = _glob_enc(dt, glob_attr, wd1_, bd1_, wd2_, bd2_, wg1_, bg1_, wg2_, bg2_)
    nemb, tab0 = _encode(state, node_attr, batch2, glob0,
                         ws1, bs1, ws2, bs2, wn1, bn1, wn2, bn2,
                         l0["wsrc"], l0["wdst"], l0["w1g"], l0["b1"])

    edges0, esums, ecnts = [], [], []
    for h in range(NH):
        gs, gd = _sc_gather(tab0, src3[h], dst3[h])
        e, es, ec = _edge0(gs, gd, ea_h[h], src2[h], lo, hi,
                           we1, be1, we2, be2, l0["w1e"], l0["w2e"], l0["b2e"])
        edges0.append(e)
        esums.append(es)
        ecnts.append(ec)
    agg0 = [_sc_scatter(edges0[h], dst3[h], zeros) for h in range(NH)]
    nemb1, nsum, ncnt = _node0(nemb, agg0[0], agg0[1], batch2, glob0,
                               l0["wvn"], l0["wva"], l0["wvg"], l0["bv1"],
                               l0["wv2"], l0["bv2"])
    tab1, glob1 = _globpre(nemb1, batch2, glob0, nsum, ncnt,
                           esums[0], ecnts[0], esums[1], ecnts[1],
                           l0["wgn"], l0["wge"], l0["wgg"], l0["bg1"],
                           l0["wg2"], l0["bg2"],
                           l1["wsrc"], l1["wdst"], l1["w1g"], l1["b1"])

    edges1 = []
    for h in range(NH):
        gs, gd = _sc_gather(tab1, src3[h], dst3[h])
        edges1.append(_edge1(gs, gd, edges0[h], l1["w1e"], l1["w2e"], l1["b2e"]))
    agg1 = [_sc_scatter(edges1[h], dst3[h], zeros) for h in range(NH)]
    out = _node1(nemb1, agg1[0], agg1[1], batch2, glob1, state,
                 l1["wvn"], l1["wva"], l1["wvg"], l1["bv1"],
                 l1["wv2"], l1["bv2"], wdec1, bdec1, wdec2, bdec2)
    return out


# speedup vs baseline: 5.5803x; 1.1951x over previous
import functools

import jax
import jax.numpy as jnp
from jax import lax
from jax.experimental import pallas as pl
from jax.experimental.pallas import tpu as pltpu
from jax.experimental.pallas import tpu_sc as plsc

N = 10000
E = 320000
B = 8
SD = 8
EDGE_A = 4
D = 64
ND = 128
GD = 128

NH = 2
EH = E // NH
NC = 2
NS = 16
NW = NC * NS
EW = EH // NW
CH = 40
NCHUNK = EW // CH
NZ = 624
NTAIL = N - NS * NZ

NBN = 1000
NBE = 4000

_SQRT_HALF = 0.7071067811865476


def _gelu(x):
    return x * 0.5 * (1.0 + lax.erf(x * _SQRT_HALF))


def _dot(a, b):
    return jnp.dot(a, b, preferred_element_type=jnp.float32)


def _mlp2(x, w1, b1, w2, b2):
    h = _gelu(_dot(x, w1[...]) + b1[...])
    return _gelu(_dot(h, w2[...]) + b2[...])


def _onehot_eq(batch_col, nb=B):
    g = lax.broadcasted_iota(jnp.int32, (1, nb), 1)
    return (batch_col == g).astype(jnp.float32)


def _onehot_range(idx_col, lo, hi):
    return ((idx_col >= lo[...]) & (idx_col < hi[...])).astype(jnp.float32)


def _fixed(shape):
    return pl.BlockSpec(shape, lambda i: (0,) * len(shape))


def _k_glob_enc(dt_ref, ga_ref, wd1, bd1, wd2, bd2, wg1, bg1, wg2, bg2, out_ref):
    h = _gelu(dt_ref[...] * wd1[...] + bd1[...])
    de = _gelu(_dot(h, wd2[...]) + bd2[...])
    ge = _mlp2(ga_ref[...], wg1, bg1, wg2, bg2)
    out_ref[...] = jnp.concatenate([de, ge], axis=1)


def _k_encode(st_ref, na_ref, b2_ref, glob_ref,
              ws1, bs1, ws2, bs2, wn1, bn1, wn2, bn2,
              wsrc, wdst, w1g, b1,
              nemb_ref, tab_ref):
    se = _mlp2(st_ref[...], ws1, bs1, ws2, bs2)
    ncoef = _mlp2(na_ref[...], wn1, bn1, wn2, bn2)
    nemb = jnp.concatenate([se, ncoef], axis=1)
    nemb_ref[...] = nemb
    gc = _dot(glob_ref[...], w1g[...])
    oh = _onehot_eq(b2_ref[...])
    a2 = _dot(nemb, wsrc[...]) + _dot(oh, gc) + b1[...]
    bd = _dot(nemb, wdst[...])
    tab_ref[...] = jnp.concatenate([a2, bd], axis=1)


def _k_edge0(gs_ref, gd_ref, ea_ref, src_ref, lo, hi,
             we1, be1, we2, be2, w1e, w2e, b2e,
             out_ref, esum_ref, ecnt_ref):
    i = pl.program_id(0)
    ee = _mlp2(ea_ref[...], we1, be1, we2, be2)
    pre = gs_ref[:, :D] + gd_ref[:, D:] + _dot(ee, w1e[...])
    out = _gelu(_dot(_gelu(pre), w2e[...]) + b2e[...])
    out_ref[...] = jnp.concatenate([out, jnp.zeros_like(out)], axis=1)
    oh = _onehot_range(src_ref[...], lo, hi)
    bs = lax.dot_general(oh, out, (((0,), (0,)), ((), ())),
                         preferred_element_type=jnp.float32)
    bc = lax.dot_general(oh, jnp.ones_like(out), (((0,), (0,)), ((), ())),
                         preferred_element_type=jnp.float32)

    @pl.when(i == 0)
    def _():
        esum_ref[...] = jnp.zeros_like(esum_ref)
        ecnt_ref[...] = jnp.zeros_like(ecnt_ref)

    esum_ref[...] += bs
    ecnt_ref[...] += bc


def _k_edge1(gs_ref, gd_ref, ee_ref, w1e, w2e, b2e, out_ref):
    pre = gs_ref[:, :D] + gd_ref[:, D:] + _dot(ee_ref[:, :D], w1e[...])
    out = _gelu(_dot(_gelu(pre), w2e[...]) + b2e[...])
    out_ref[...] = jnp.concatenate([out, jnp.zeros_like(out)], axis=1)


def _node_new(nemb_ref, agg1_ref, agg2_ref, b2_ref, glob_ref,
              wvn, wva, wvg, bv1, wv2, bv2):
    agg = (agg1_ref[0, :, :D] + agg1_ref[1, :, :D]
           + agg2_ref[0, :, :D] + agg2_ref[1, :, :D])
    gv = _dot(glob_ref[...], wvg[...])
    oh = _onehot_eq(b2_ref[...])
    pre = _dot(nemb_ref[...], wvn[...]) + _dot(agg, wva[...]) + _dot(oh, gv) + bv1[...]
    return _gelu(_dot(_gelu(pre), wv2[...]) + bv2[...]), oh


def _k_node0(nemb_ref, agg1_ref, agg2_ref, b2_ref, glob_ref,
             wvn, wva, wvg, bv1, wv2, bv2,
             nnew_ref, nsum_ref, ncnt_ref):
    i = pl.program_id(0)
    nnew, oh = _node_new(nemb_ref, agg1_ref, agg2_ref, b2_ref, glob_ref,
                         wvn, wva, wvg, bv1, wv2, bv2)
    nnew_ref[...] = nnew
    bs = lax.dot_general(oh, nnew, (((0,), (0,)), ((), ())),
                         preferred_element_type=jnp.float32)
    bc = lax.dot_general(oh, jnp.ones_like(nnew), (((0,), (0,)), ((), ())),
                         preferred_element_type=jnp.float32)

    @pl.when(i == 0)
    def _():
        nsum_ref[...] = jnp.zeros_like(nsum_ref)
        ncnt_ref[...] = jnp.zeros_like(ncnt_ref)

    nsum_ref[...] += bs
    ncnt_ref[...] += bc


def _k_node1(nemb_ref, agg1_ref, agg2_ref, b2_ref, glob_ref, st_ref,
             wvn, wva, wvg, bv1, wv2, bv2, wd1, bd1, wd2, bd2,
             out_ref):
    nnew, _ = _node_new(nemb_ref, agg1_ref, agg2_ref, b2_ref, glob_ref,
                        wvn, wva, wvg, bv1, wv2, bv2)
    h = _gelu(_dot(nnew, wd1[...]) + bd1[...])
    out_ref[...] = st_ref[...] + _dot(h, wd2[...]) + bd2[...]


def _k_globpre(nemb_ref, b2_ref, glob_ref, nsum, ncnt, esum1, ecnt1, esum2, ecnt2,
               wgn, wge, wgg, bg1, wg2, bg2, wsrc, wdst, w1g, b1,
               tab_ref, glob1_ref):
    nmean = nsum[...] / jnp.maximum(ncnt[...], 1.0)
    emean = (esum1[...] + esum2[...]) / jnp.maximum(ecnt1[...] + ecnt2[...], 1.0)
    gpre = _dot(nmean, wgn[...]) + _dot(emean, wge[...]) + _dot(glob_ref[...], wgg[...]) + bg1[...]
    glob1 = _gelu(_dot(_gelu(gpre), wg2[...]) + bg2[...])
    glob1_ref[...] = glob1
    gc = _dot(glob1, w1g[...])
    oh = _onehot_eq(b2_ref[...])
    a2 = _dot(nemb_ref[...], wsrc[...]) + _dot(oh, gc) + b1[...]
    bd = _dot(nemb_ref[...], wdst[...])
    tab_ref[...] = jnp.concatenate([a2, bd], axis=1)


_TC_PARAMS = pltpu.CompilerParams(dimension_semantics=("arbitrary",))


def _glob_enc(dt, glob_attr, wd1, bd1, wd2, bd2, wg1, bg1, wg2, bg2):
    return pl.pallas_call(
        _k_glob_enc,
        out_shape=jax.ShapeDtypeStruct((B, GD), jnp.float32),
    )(dt, glob_attr, wd1, bd1, wd2, bd2, wg1, bg1, wg2, bg2)


def _encode(state, node_attr, batch2, glob0, *ws):
    grid = (N // NBN,)
    return pl.pallas_call(
        _k_encode,
        grid=grid,
        in_specs=[
            pl.BlockSpec((NBN, SD), lambda i: (i, 0)),
            pl.BlockSpec((NBN, SD), lambda i: (i, 0)),
            pl.BlockSpec((NBN, 1), lambda i: (i, 0)),
            _fixed((B, GD)),
        ] + [_fixed(w.shape) for w in ws],
        out_specs=[
            pl.BlockSpec((NBN, ND), lambda i: (i, 0)),
            pl.BlockSpec((NBN, 2 * D), lambda i: (i, 0)),
        ],
        out_shape=[
            jax.ShapeDtypeStruct((N, ND), jnp.float32),
            jax.ShapeDtypeStruct((N, 2 * D), jnp.float32),
        ],
        compiler_params=_TC_PARAMS,
    )(state, node_attr, batch2, glob0, *ws)


def _edge0(gs, gd, edge_attr, src2, lo, hi, *ws):
    grid = (EH // NBE,)
    return pl.pallas_call(
        _k_edge0,
        grid=grid,
        in_specs=[
            pl.BlockSpec((NBE, 2 * D), lambda i: (i, 0)),
            pl.BlockSpec((NBE, 2 * D), lambda i: (i, 0)),
            pl.BlockSpec((NBE, 4), lambda i: (i, 0)),
            pl.BlockSpec((NBE, 1), lambda i: (i, 0)),
            _fixed((1, B)),
            _fixed((1, B)),
        ] + [_fixed(w.shape) for w in ws],
        out_specs=[
            pl.BlockSpec((NBE, 2 * D), lambda i: (i, 0)),
            _fixed((B, D)),
            _fixed((B, D)),
        ],
        out_shape=[
            jax.ShapeDtypeStruct((EH, 2 * D), jnp.float32),
            jax.ShapeDtypeStruct((B, D), jnp.float32),
            jax.ShapeDtypeStruct((B, D), jnp.float32),
        ],
        compiler_params=_TC_PARAMS,
    )(gs, gd, edge_attr, src2, lo, hi, *ws)


def _edge1(gs, gd, ee, *ws):
    grid = (EH // NBE,)
    return pl.pallas_call(
        _k_edge1,
        grid=grid,
        in_specs=[
            pl.BlockSpec((NBE, 2 * D), lambda i: (i, 0)),
            pl.BlockSpec((NBE, 2 * D), lambda i: (i, 0)),
            pl.BlockSpec((NBE, 2 * D), lambda i: (i, 0)),
        ] + [_fixed(w.shape) for w in ws],
        out_specs=pl.BlockSpec((NBE, 2 * D), lambda i: (i, 0)),
        out_shape=jax.ShapeDtypeStruct((EH, 2 * D), jnp.float32),
        compiler_params=_TC_PARAMS,
    )(gs, gd, ee, *ws)


def _node0(nemb, agg1, agg2, batch2, glob0, *ws):
    grid = (N // NBN,)
    return pl.pallas_call(
        _k_node0,
        grid=grid,
        in_specs=[
            pl.BlockSpec((NBN, ND), lambda i: (i, 0)),
            pl.BlockSpec((NC, NBN, 2 * D), lambda i: (0, i, 0)),
            pl.BlockSpec((NC, NBN, 2 * D), lambda i: (0, i, 0)),
            pl.BlockSpec((NBN, 1), lambda i: (i, 0)),
            _fixed((B, GD)),
        ] + [_fixed(w.shape) for w in ws],
        out_specs=[
            pl.BlockSpec((NBN, ND), lambda i: (i, 0)),
            _fixed((B, ND)),
            _fixed((B, ND)),
        ],
        out_shape=[
            jax.ShapeDtypeStruct((N, ND), jnp.float32),
            jax.ShapeDtypeStruct((B, ND), jnp.float32),
            jax.ShapeDtypeStruct((B, ND), jnp.float32),
        ],
        compiler_params=_TC_PARAMS,
    )(nemb, agg1, agg2, batch2, glob0, *ws)


def _node1(nemb, agg1, agg2, batch2, glob1, state, *ws):
    grid = (N // NBN,)
    return pl.pallas_call(
        _k_node1,
        grid=grid,
        in_specs=[
            pl.BlockSpec((NBN, ND), lambda i: (i, 0)),
            pl.BlockSpec((NC, NBN, 2 * D), lambda i: (0, i, 0)),
            pl.BlockSpec((NC, NBN, 2 * D), lambda i: (0, i, 0)),
            pl.BlockSpec((NBN, 1), lambda i: (i, 0)),
            _fixed((B, GD)),
            pl.BlockSpec((NBN, SD), lambda i: (i, 0)),
        ] + [_fixed(w.shape) for w in ws],
        out_specs=pl.BlockSpec((NBN, SD), lambda i: (i, 0)),
        out_shape=jax.ShapeDtypeStruct((N, SD), jnp.float32),
        compiler_params=_TC_PARAMS,
    )(nemb, agg1, agg2, batch2, glob1, state, *ws)


def _globpre(nemb1, batch2, glob0, nsum, ncnt, esum1, ecnt1, esum2, ecnt2, *ws):
    grid = (N // NBN,)
    return pl.pallas_call(
        _k_globpre,
        grid=grid,
        in_specs=[
            pl.BlockSpec((NBN, ND), lambda i: (i, 0)),
            pl.BlockSpec((NBN, 1), lambda i: (i, 0)),
            _fixed((B, GD)),
            _fixed((B, ND)),
            _fixed((B, ND)),
            _fixed((B, D)),
            _fixed((B, D)),
            _fixed((B, D)),
            _fixed((B, D)),
        ] + [_fixed(w.shape) for w in ws],
        out_specs=[
            pl.BlockSpec((NBN, 2 * D), lambda i: (i, 0)),
            _fixed((B, GD)),
        ],
        out_shape=[
            jax.ShapeDtypeStruct((N, 2 * D), jnp.float32),
            jax.ShapeDtypeStruct((B, GD), jnp.float32),
        ],
        compiler_params=_TC_PARAMS,
    )(nemb1, batch2, glob0, nsum, ncnt, esum1, ecnt1, esum2, ecnt2, *ws)


def _sc_gather_body(tab_hbm, src_hbm, dst_hbm, ga_hbm, gb_hbm,
                    idx_a, idx_b, ra0, rb0, ra1, rb1,
                    ga0, gb0, ga1, gb1, wa0, wb0, wa1, wb1):
    c = lax.axis_index("c")
    s = lax.axis_index("s")
    wid = s * NC + c
    base = wid * EW
    pltpu.sync_copy(src_hbm.at[wid], idx_a)
    pltpu.sync_copy(dst_hbm.at[wid], idx_b)

    def g_start(j, ra, rb, sa, sb):
        pltpu.async_copy(tab_hbm.at[idx_a.at[j]], ra, sa)
        pltpu.async_copy(tab_hbm.at[idx_b.at[j]], rb, sb)

    def g_wait(j, ra, rb, sa, sb):
        pltpu.make_async_copy(tab_hbm.at[idx_a.at[j]], ra, sa).wait()
        pltpu.make_async_copy(tab_hbm.at[idx_b.at[j]], rb, sb).wait()

    def w_start(j, ra, rb, sa, sb):
        pltpu.async_copy(ra, ga_hbm.at[pl.ds(base + j * CH, CH)], sa)
        pltpu.async_copy(rb, gb_hbm.at[pl.ds(base + j * CH, CH)], sb)

    def w_wait(j, ra, rb, sa, sb):
        pltpu.make_async_copy(ra, ga_hbm.at[pl.ds(base + j * CH, CH)], sa).wait()
        pltpu.make_async_copy(rb, gb_hbm.at[pl.ds(base + j * CH, CH)], sb).wait()

    g_start(0, ra0, rb0, ga0, gb0)
    g_start(1, ra1, rb1, ga1, gb1)

    def body(t, carry):
        j0 = 2 * t
        j1 = j0 + 1
        j2 = j0 + 2
        j3 = j0 + 3
        g_wait(j0, ra0, rb0, ga0, gb0)
        w_start(j0, ra0, rb0, wa0, wb0)
        g_wait(j1, ra1, rb1, ga1, gb1)
        w_start(j1, ra1, rb1, wa1, wb1)
        w_wait(j0, ra0, rb0, wa0, wb0)

        @pl.when(j2 < NCHUNK)
        def _():
            g_start(j2, ra0, rb0, ga0, gb0)

        w_wait(j1, ra1, rb1, wa1, wb1)

        @pl.when(j3 < NCHUNK)
        def _():
            g_start(j3, ra1, rb1, ga1, gb1)

        return carry

    lax.fori_loop(0, NCHUNK // 2, body, 0)
    jt = NCHUNK - 1
    g_wait(jt, ra0, rb0, ga0, gb0)
    pltpu.sync_copy(ra0, ga_hbm.at[pl.ds(base + jt * CH, CH)])
    pltpu.sync_copy(rb0, gb_hbm.at[pl.ds(base + jt * CH, CH)])


def _sc_scatter_body(e_hbm, dst_hbm, zero_hbm, out_hbm, idx_v, rows0, rows1,
                     acc, rd0, rd1, sc0, sc1):
    c = lax.axis_index("c")
    s = lax.axis_index("s")
    wid = s * NC + c
    pltpu.sync_copy(zero_hbm.at[pl.ds(0, NZ)], acc.at[pl.ds(s * NZ, NZ)])

    @pl.when(s == NS - 1)
    def _():
        pltpu.sync_copy(zero_hbm.at[pl.ds(0, NTAIL)], acc.at[pl.ds(NS * NZ, NTAIL)])

    plsc.subcore_barrier()
    pltpu.sync_copy(dst_hbm.at[wid], idx_v)
    base = wid * EW

    def r_start(j, rows, sem):
        pltpu.async_copy(e_hbm.at[pl.ds(base + j * CH, CH)], rows, sem)

    def r_wait(j, rows, sem):
        pltpu.make_async_copy(e_hbm.at[pl.ds(base + j * CH, CH)], rows, sem).wait()

    def s_start(j, rows, sem):
        pltpu.async_copy(rows, acc.at[idx_v.at[j]], sem, add=True)

    def s_wait(j, rows, sem):
        pltpu.make_async_copy(rows, acc.at[idx_v.at[j]], sem).wait()

    r_start(0, rows0, rd0)
    r_start(1, rows1, rd1)

    def body(t, carry):
        j0 = 2 * t
        j1 = j0 + 1
        j2 = j0 + 2
        j3 = j0 + 3
        r_wait(j0, rows0, rd0)
        s_start(j0, rows0, sc0)
        r_wait(j1, rows1, rd1)
        s_start(j1, rows1, sc1)
        s_wait(j0, rows0, sc0)

        @pl.when(j2 < NCHUNK)
        def _():
            r_start(j2, rows0, rd0)

        s_wait(j1, rows1, sc1)

        @pl.when(j3 < NCHUNK)
        def _():
            r_start(j3, rows1, rd1)

        return carry

    lax.fori_loop(0, NCHUNK // 2, body, 0)
    jt = NCHUNK - 1
    r_wait(jt, rows0, rd0)
    pltpu.sync_copy(rows0, acc.at[idx_v.at[jt]], add=True)
    plsc.subcore_barrier()
    pltpu.sync_copy(acc.at[pl.ds(s * NZ, NZ)], out_hbm.at[c, pl.ds(s * NZ, NZ)])

    @pl.when(s == NS - 1)
    def _():
        pltpu.sync_copy(acc.at[pl.ds(NS * NZ, NTAIL)],
                        out_hbm.at[c, pl.ds(NS * NZ, NTAIL)])


@functools.cache
def _sc_kernels():
    mesh = plsc.VectorSubcoreMesh(core_axis_name="c", subcore_axis_name="s",
                                  num_cores=NC, num_subcores=NS)
    gather = pl.kernel(
        _sc_gather_body,
        out_type=(jax.ShapeDtypeStruct((EH, 2 * D), jnp.float32),
                  jax.ShapeDtypeStruct((EH, 2 * D), jnp.float32)),
        mesh=mesh,
        scratch_types=[
            pltpu.VMEM((NCHUNK, CH), jnp.int32),
            pltpu.VMEM((NCHUNK, CH), jnp.int32),
            pltpu.VMEM((CH, 2 * D), jnp.float32),
            pltpu.VMEM((CH, 2 * D), jnp.float32),
            pltpu.VMEM((CH, 2 * D), jnp.float32),
            pltpu.VMEM((CH, 2 * D), jnp.float32),
        ] + [pltpu.SemaphoreType.DMA] * 8,
    )
    scatter = pl.kernel(
        _sc_scatter_body,
        out_type=jax.ShapeDtypeStruct((NC, N, 2 * D), jnp.float32),
        mesh=mesh,
        scratch_types=[
            pltpu.VMEM((NCHUNK, CH), jnp.int32),
            pltpu.VMEM((CH, 2 * D), jnp.float32),
            pltpu.VMEM((CH, 2 * D), jnp.float32),
            pltpu.VMEM_SHARED((N, 2 * D), jnp.float32),
        ] + [pltpu.SemaphoreType.DMA] * 4,
    )
    return gather, scatter


def _sc_gather(tab, src3, dst3):
    return _sc_kernels()[0](tab, src3, dst3)


def _sc_scatter(edge_emb, dst3, zeros):
    return _sc_kernels()[1](edge_emb, dst3, zeros)


def _lin_w(p):
    return p["W"], p["b"].reshape(1, -1)


def kernel(state, dt, node_attr, edge_attr, glob_attr, params, edge_index, batch, ptr):
    src = edge_index[0]
    dst = edge_index[1]
    src3 = src.reshape(NH, NW, NCHUNK, CH)
    dst3 = dst.reshape(NH, NW, NCHUNK, CH)
    src2 = src.reshape(NH, EH, 1)
    ea_h = edge_attr.reshape(NH, EH, EDGE_A)
    batch2 = batch.reshape(N, 1)
    lo = ptr[:B].reshape(1, B)
    hi = ptr[1:B + 1].reshape(1, B)
    zeros = jnp.zeros((NZ, 2 * D), jnp.float32)

    ws1, bs1 = _lin_w(params["state_enc"]["l1"])
    ws2, bs2 = _lin_w(params["state_enc"]["l2"])
    wn1, bn1 = _lin_w(params["node_enc"]["l1"])
    wn2, bn2 = _lin_w(params["node_enc"]["l2"])
    wd1_, bd1_ = _lin_w(params["dt_enc"]["l1"])
    wd2_, bd2_ = _lin_w(params["dt_enc"]["l2"])
    wg1_, bg1_ = _lin_w(params["glob_enc"]["l1"])
    wg2_, bg2_ = _lin_w(params["glob_enc"]["l2"])
    we1, be1 = _lin_w(params["edge_enc"]["l1"])
    we2, be2 = _lin_w(params["edge_enc"]["l2"])
    wdec1, bdec1 = _lin_w(params["dec"]["l1"])
    wdec2, bdec2 = _lin_w(params["dec"]["l2"])

    layers = []
    for layer in params["gn"]:
        w1, b1 = _lin_w(layer["phi_e"]["l1"])
        w2e, b2e = _lin_w(layer["phi_e"]["l2"])
        wv1, bv1 = _lin_w(layer["phi_v"]["l1"])
        wv2, bv2 = _lin_w(layer["phi_v"]["l2"])
        wg1, bg1 = _lin_w(layer["phi_g"]["l1"])
        wg2, bg2 = _lin_w(layer["phi_g"]["l2"])
        layers.append(dict(
            wsrc=w1[:ND], wdst=w1[ND:2 * ND], w1e=w1[2 * ND:2 * ND + D],
            w1g=w1[2 * ND + D:], b1=b1, w2e=w2e, b2e=b2e,
            wvn=wv1[:ND], wva=wv1[ND:ND + D], wvg=wv1[ND + D:], bv1=bv1,
            wv2=wv2, bv2=bv2,
            wgn=wg1[:ND], wge=wg1[ND:ND + D], wgg=wg1[ND + D:], bg1=bg1,
            wg2=wg2, bg2=bg2,
        ))
    l0, l1 = layers

    glob0 = _glob_enc(dt, glob_attr, wd1_, bd1_, wd2_, bd2_, wg1_, bg1_, wg2_, bg2_)
    nemb, tab0 = _encode(state, node_attr, batch2, glob0,
                         ws1, bs1, ws2, bs2, wn1, bn1, wn2, bn2,
                         l0["wsrc"], l0["wdst"], l0["w1g"], l0["b1"])

    edges0, esums, ecnts = [], [], []
    for h in range(NH):
        gs, gd = _sc_gather(tab0, src3[h], dst3[h])
        e, es, ec = _edge0(gs, gd, ea_h[h], src2[h], lo, hi,
                           we1, be1, we2, be2, l0["w1e"], l0["w2e"], l0["b2e"])
        edges0.append(e)
        esums.append(es)
        ecnts.append(ec)
    agg0 = [_sc_scatter(edges0[h], dst3[h], zeros) for h in range(NH)]
    nemb1, nsum, ncnt = _node0(nemb, agg0[0], agg0[1], batch2, glob0,
                               l0["wvn"], l0["wva"], l0["wvg"], l0["bv1"],
                               l0["wv2"], l0["bv2"])
    tab1, glob1 = _globpre(nemb1, batch2, glob0, nsum, ncnt,
                           esums[0], ecnts[0], esums[1], ecnts[1],
                           l0["wgn"], l0["wge"], l0["wgg"], l0["bg1"],
                           l0["wg2"], l0["bg2"],
                           l1["wsrc"], l1["wdst"], l1["w1g"], l1["b1"])

    edges1 = []
    for h in range(NH):
        gs, gd = _sc_gather(tab1, src3[h], dst3[h])
        edges1.append(_edge1(gs, gd, edges0[h], l1["w1e"], l1["w2e"], l1["b2e"]))
    agg1 = [_sc_scatter(edges1[h], dst3[h], zeros) for h in range(NH)]
    out = _node1(nemb1, agg1[0], agg1[1], batch2, glob1, state,
                 l1["wvn"], l1["wva"], l1["wvg"], l1["bv1"],
                 l1["wv2"], l1["bv2"], wdec1, bdec1, wdec2, bdec2)
    return out


# speedup vs baseline: 6.1563x; 1.1032x over previous
import functools

import jax
import jax.numpy as jnp
from jax import lax
from jax.experimental import pallas as pl
from jax.experimental.pallas import tpu as pltpu
from jax.experimental.pallas import tpu_sc as plsc

N = 10000
E = 320000
B = 8
SD = 8
EDGE_A = 4
D = 64
ND = 128
GD = 128

NH = 2
EH = E // NH
NC = 2
NS = 16
NW = NC * NS
EW = EH // NW
CH = 40
NCHUNK = EW // CH
NZ = 624
NTAIL = N - NS * NZ

NBN = 1000
NBE = 4000

_SQRT_HALF = 0.7071067811865476


def _gelu(x):
    return x * 0.5 * (1.0 + lax.erf(x * _SQRT_HALF))


def _dot(a, b):
    return jnp.dot(a, b, preferred_element_type=jnp.float32)


def _mlp2(x, w1, b1, w2, b2):
    h = _gelu(_dot(x, w1[...]) + b1[...])
    return _gelu(_dot(h, w2[...]) + b2[...])


def _onehot_eq(batch_col, nb=B):
    g = lax.broadcasted_iota(jnp.int32, (1, nb), 1)
    return (batch_col == g).astype(jnp.float32)


def _onehot_range(idx_col, lo, hi):
    return ((idx_col >= lo[...]) & (idx_col < hi[...])).astype(jnp.float32)


def _fixed(shape):
    return pl.BlockSpec(shape, lambda i: (0,) * len(shape))


def _k_glob_enc(dt_ref, ga_ref, wd1, bd1, wd2, bd2, wg1, bg1, wg2, bg2, out_ref):
    h = _gelu(dt_ref[...] * wd1[...] + bd1[...])
    de = _gelu(_dot(h, wd2[...]) + bd2[...])
    ge = _mlp2(ga_ref[...], wg1, bg1, wg2, bg2)
    out_ref[...] = jnp.concatenate([de, ge], axis=1)


def _k_encode(st_ref, na_ref, b2_ref, glob_ref,
              ws1, bs1, ws2, bs2, wn1, bn1, wn2, bn2,
              wsrc, wdst, w1g, b1,
              nemb_ref, tab_ref):
    se = _mlp2(st_ref[...], ws1, bs1, ws2, bs2)
    ncoef = _mlp2(na_ref[...], wn1, bn1, wn2, bn2)
    nemb = jnp.concatenate([se, ncoef], axis=1)
    nemb_ref[...] = nemb
    gc = _dot(glob_ref[...], w1g[...])
    oh = _onehot_eq(b2_ref[...])
    a2 = _dot(nemb, wsrc[...]) + _dot(oh, gc) + b1[...]
    bd = _dot(nemb, wdst[...])
    tab_ref[...] = jnp.concatenate([a2, bd], axis=1)


def _k_edge0(gc_ref, ea_ref, src_ref, lo, hi,
             we1, be1, we2, be2, w1e, w2e, b2e,
             out_ref, esum_ref, ecnt_ref):
    i = pl.program_id(0)
    ee = _mlp2(ea_ref[...], we1, be1, we2, be2)
    pre = gc_ref[...] + _dot(ee, w1e[...])
    out = _gelu(_dot(_gelu(pre), w2e[...]) + b2e[...])
    out_ref[...] = jnp.concatenate([out, jnp.zeros_like(out)], axis=1)
    oh = _onehot_range(src_ref[...], lo, hi)
    bs = lax.dot_general(oh, out, (((0,), (0,)), ((), ())),
                         preferred_element_type=jnp.float32)
    bc = lax.dot_general(oh, jnp.ones_like(out), (((0,), (0,)), ((), ())),
                         preferred_element_type=jnp.float32)

    @pl.when(i == 0)
    def _():
        esum_ref[...] = jnp.zeros_like(esum_ref)
        ecnt_ref[...] = jnp.zeros_like(ecnt_ref)

    esum_ref[...] += bs
    ecnt_ref[...] += bc


def _k_edge1(gc_ref, ee_ref, w1e, w2e, b2e, out_ref):
    pre = gc_ref[...] + _dot(ee_ref[:, :D], w1e[...])
    out = _gelu(_dot(_gelu(pre), w2e[...]) + b2e[...])
    out_ref[...] = jnp.concatenate([out, jnp.zeros_like(out)], axis=1)


def _node_new(nemb_ref, agg1_ref, agg2_ref, b2_ref, glob_ref,
              wvn, wva, wvg, bv1, wv2, bv2):
    agg = (agg1_ref[0, :, :D] + agg1_ref[1, :, :D]
           + agg2_ref[0, :, :D] + agg2_ref[1, :, :D])
    gv = _dot(glob_ref[...], wvg[...])
    oh = _onehot_eq(b2_ref[...])
    pre = _dot(nemb_ref[...], wvn[...]) + _dot(agg, wva[...]) + _dot(oh, gv) + bv1[...]
    return _gelu(_dot(_gelu(pre), wv2[...]) + bv2[...]), oh


def _k_node0(nemb_ref, agg1_ref, agg2_ref, b2_ref, glob_ref,
             wvn, wva, wvg, bv1, wv2, bv2,
             nnew_ref, nsum_ref, ncnt_ref):
    i = pl.program_id(0)
    nnew, oh = _node_new(nemb_ref, agg1_ref, agg2_ref, b2_ref, glob_ref,
                         wvn, wva, wvg, bv1, wv2, bv2)
    nnew_ref[...] = nnew
    bs = lax.dot_general(oh, nnew, (((0,), (0,)), ((), ())),
                         preferred_element_type=jnp.float32)
    bc = lax.dot_general(oh, jnp.ones_like(nnew), (((0,), (0,)), ((), ())),
                         preferred_element_type=jnp.float32)

    @pl.when(i == 0)
    def _():
        nsum_ref[...] = jnp.zeros_like(nsum_ref)
        ncnt_ref[...] = jnp.zeros_like(ncnt_ref)

    nsum_ref[...] += bs
    ncnt_ref[...] += bc


def _k_node1(nemb_ref, agg1_ref, agg2_ref, b2_ref, glob_ref, st_ref,
             wvn, wva, wvg, bv1, wv2, bv2, wd1, bd1, wd2, bd2,
             out_ref):
    nnew, _ = _node_new(nemb_ref, agg1_ref, agg2_ref, b2_ref, glob_ref,
                        wvn, wva, wvg, bv1, wv2, bv2)
    h = _gelu(_dot(nnew, wd1[...]) + bd1[...])
    out_ref[...] = st_ref[...] + _dot(h, wd2[...]) + bd2[...]


def _k_globpre(nemb_ref, b2_ref, glob_ref, nsum, ncnt, esum1, ecnt1, esum2, ecnt2,
               wgn, wge, wgg, bg1, wg2, bg2, wsrc, wdst, w1g, b1,
               tab_ref, glob1_ref):
    nmean = nsum[...] / jnp.maximum(ncnt[...], 1.0)
    emean = (esum1[...] + esum2[...]) / jnp.maximum(ecnt1[...] + ecnt2[...], 1.0)
    gpre = _dot(nmean, wgn[...]) + _dot(emean, wge[...]) + _dot(glob_ref[...], wgg[...]) + bg1[...]
    glob1 = _gelu(_dot(_gelu(gpre), wg2[...]) + bg2[...])
    glob1_ref[...] = glob1
    gc = _dot(glob1, w1g[...])
    oh = _onehot_eq(b2_ref[...])
    a2 = _dot(nemb_ref[...], wsrc[...]) + _dot(oh, gc) + b1[...]
    bd = _dot(nemb_ref[...], wdst[...])
    tab_ref[...] = jnp.concatenate([a2, bd], axis=1)


_TC_PARAMS = pltpu.CompilerParams(dimension_semantics=("arbitrary",))


def _glob_enc(dt, glob_attr, wd1, bd1, wd2, bd2, wg1, bg1, wg2, bg2):
    return pl.pallas_call(
        _k_glob_enc,
        out_shape=jax.ShapeDtypeStruct((B, GD), jnp.float32),
    )(dt, glob_attr, wd1, bd1, wd2, bd2, wg1, bg1, wg2, bg2)


def _encode(state, node_attr, batch2, glob0, *ws):
    grid = (N // NBN,)
    return pl.pallas_call(
        _k_encode,
        grid=grid,
        in_specs=[
            pl.BlockSpec((NBN, SD), lambda i: (i, 0)),
            pl.BlockSpec((NBN, SD), lambda i: (i, 0)),
            pl.BlockSpec((NBN, 1), lambda i: (i, 0)),
            _fixed((B, GD)),
        ] + [_fixed(w.shape) for w in ws],
        out_specs=[
            pl.BlockSpec((NBN, ND), lambda i: (i, 0)),
            pl.BlockSpec((NBN, 2 * D), lambda i: (i, 0)),
        ],
        out_shape=[
            jax.ShapeDtypeStruct((N, ND), jnp.float32),
            jax.ShapeDtypeStruct((N, 2 * D), jnp.float32),
        ],
        compiler_params=_TC_PARAMS,
    )(state, node_attr, batch2, glob0, *ws)


def _edge0(gc, edge_attr, src2, lo, hi, *ws):
    grid = (EH // NBE,)
    return pl.pallas_call(
        _k_edge0,
        grid=grid,
        in_specs=[
            pl.BlockSpec((NBE, D), lambda i: (i, 0)),
            pl.BlockSpec((NBE, 4), lambda i: (i, 0)),
            pl.BlockSpec((NBE, 1), lambda i: (i, 0)),
            _fixed((1, B)),
            _fixed((1, B)),
        ] + [_fixed(w.shape) for w in ws],
        out_specs=[
            pl.BlockSpec((NBE, 2 * D), lambda i: (i, 0)),
            _fixed((B, D)),
            _fixed((B, D)),
        ],
        out_shape=[
            jax.ShapeDtypeStruct((EH, 2 * D), jnp.float32),
            jax.ShapeDtypeStruct((B, D), jnp.float32),
            jax.ShapeDtypeStruct((B, D), jnp.float32),
        ],
        compiler_params=_TC_PARAMS,
    )(gc, edge_attr, src2, lo, hi, *ws)


def _edge1(gc, ee, *ws):
    grid = (EH // NBE,)
    return pl.pallas_call(
        _k_edge1,
        grid=grid,
        in_specs=[
            pl.BlockSpec((NBE, D), lambda i: (i, 0)),
            pl.BlockSpec((NBE, 2 * D), lambda i: (i, 0)),
        ] + [_fixed(w.shape) for w in ws],
        out_specs=pl.BlockSpec((NBE, 2 * D), lambda i: (i, 0)),
        out_shape=jax.ShapeDtypeStruct((EH, 2 * D), jnp.float32),
        compiler_params=_TC_PARAMS,
    )(gc, ee, *ws)


def _node0(nemb, agg1, agg2, batch2, glob0, *ws):
    grid = (N // NBN,)
    return pl.pallas_call(
        _k_node0,
        grid=grid,
        in_specs=[
            pl.BlockSpec((NBN, ND), lambda i: (i, 0)),
            pl.BlockSpec((NC, NBN, 2 * D), lambda i: (0, i, 0)),
            pl.BlockSpec((NC, NBN, 2 * D), lambda i: (0, i, 0)),
            pl.BlockSpec((NBN, 1), lambda i: (i, 0)),
            _fixed((B, GD)),
        ] + [_fixed(w.shape) for w in ws],
        out_specs=[
            pl.BlockSpec((NBN, ND), lambda i: (i, 0)),
            _fixed((B, ND)),
            _fixed((B, ND)),
        ],
        out_shape=[
            jax.ShapeDtypeStruct((N, ND), jnp.float32),
            jax.ShapeDtypeStruct((B, ND), jnp.float32),
            jax.ShapeDtypeStruct((B, ND), jnp.float32),
        ],
        compiler_params=_TC_PARAMS,
    )(nemb, agg1, agg2, batch2, glob0, *ws)


def _node1(nemb, agg1, agg2, batch2, glob1, state, *ws):
    grid = (N // NBN,)
    return pl.pallas_call(
        _k_node1,
        grid=grid,
        in_specs=[
            pl.BlockSpec((NBN, ND), lambda i: (i, 0)),
            pl.BlockSpec((NC, NBN, 2 * D), lambda i: (0, i, 0)),
            pl.BlockSpec((NC, NBN, 2 * D), lambda i: (0, i, 0)),
            pl.BlockSpec((NBN, 1), lambda i: (i, 0)),
            _fixed((B, GD)),
            pl.BlockSpec((NBN, SD), lambda i: (i, 0)),
        ] + [_fixed(w.shape) for w in ws],
        out_specs=pl.BlockSpec((NBN, SD), lambda i: (i, 0)),
        out_shape=jax.ShapeDtypeStruct((N, SD), jnp.float32),
        compiler_params=_TC_PARAMS,
    )(nemb, agg1, agg2, batch2, glob1, state, *ws)


def _globpre(nemb1, batch2, glob0, nsum, ncnt, esum1, ecnt1, esum2, ecnt2, *ws):
    grid = (N // NBN,)
    return pl.pallas_call(
        _k_globpre,
        grid=grid,
        in_specs=[
            pl.BlockSpec((NBN, ND), lambda i: (i, 0)),
            pl.BlockSpec((NBN, 1), lambda i: (i, 0)),
            _fixed((B, GD)),
            _fixed((B, ND)),
            _fixed((B, ND)),
            _fixed((B, D)),
            _fixed((B, D)),
            _fixed((B, D)),
            _fixed((B, D)),
        ] + [_fixed(w.shape) for w in ws],
        out_specs=[
            pl.BlockSpec((NBN, 2 * D), lambda i: (i, 0)),
            _fixed((B, GD)),
        ],
        out_shape=[
            jax.ShapeDtypeStruct((N, 2 * D), jnp.float32),
            jax.ShapeDtypeStruct((B, GD), jnp.float32),
        ],
        compiler_params=_TC_PARAMS,
    )(nemb1, batch2, glob0, nsum, ncnt, esum1, ecnt1, esum2, ecnt2, *ws)


def _sc_gather_body(tab_hbm, src_hbm, dst_hbm, gc_hbm,
                    idx_a, idx_b, ra0, rb0, ra1, rb1, cb0, cb1,
                    ga0, gb0, ga1, gb1, wa0, wb0):
    c = lax.axis_index("c")
    s = lax.axis_index("s")
    wid = s * NC + c
    base = wid * EW
    pltpu.sync_copy(src_hbm.at[wid], idx_a)
    pltpu.sync_copy(dst_hbm.at[wid], idx_b)

    def g_start(j, ra, rb, sa, sb):
        pltpu.async_copy(tab_hbm.at[idx_a.at[j]], ra, sa)
        pltpu.async_copy(tab_hbm.at[idx_b.at[j]], rb, sb)

    def g_wait(j, ra, rb, sa, sb):
        pltpu.make_async_copy(tab_hbm.at[idx_a.at[j]], ra, sa).wait()
        pltpu.make_async_copy(tab_hbm.at[idx_b.at[j]], rb, sb).wait()

    def add_halves(ra, rb, comb):
        def rbody(r, carry):
            for k in range(D // 16):
                comb[r, pl.ds(k * 16, 16)] = (ra[r, pl.ds(k * 16, 16)]
                                              + rb[r, pl.ds(D + k * 16, 16)])
            return carry
        lax.fori_loop(0, CH, rbody, 0, unroll=4)

    def w_start(j, comb, sa):
        pltpu.async_copy(comb, gc_hbm.at[pl.ds(base + j * CH, CH)], sa)

    def w_wait(j, comb, sa):
        pltpu.make_async_copy(comb, gc_hbm.at[pl.ds(base + j * CH, CH)], sa).wait()

    g_start(0, ra0, rb0, ga0, gb0)
    g_start(1, ra1, rb1, ga1, gb1)

    def body(t, carry):
        j0 = 2 * t
        j1 = j0 + 1
        j2 = j0 + 2
        j3 = j0 + 3
        g_wait(j0, ra0, rb0, ga0, gb0)

        @pl.when(t > 0)
        def _():
            w_wait(j0 - 2, cb0, wa0)

        add_halves(ra0, rb0, cb0)
        w_start(j0, cb0, wa0)

        @pl.when(j2 < NCHUNK)
        def _():
            g_start(j2, ra0, rb0, ga0, gb0)

        g_wait(j1, ra1, rb1, ga1, gb1)

        @pl.when(t > 0)
        def _():
            w_wait(j1 - 2, cb1, wb0)

        add_halves(ra1, rb1, cb1)
        w_start(j1, cb1, wb0)

        @pl.when(j3 < NCHUNK)
        def _():
            g_start(j3, ra1, rb1, ga1, gb1)

        return carry

    lax.fori_loop(0, NCHUNK // 2, body, 0)
    jt = NCHUNK - 1
    g_wait(jt, ra0, rb0, ga0, gb0)
    w_wait(jt - 2, cb0, wa0)
    w_wait(jt - 1, cb1, wb0)
    add_halves(ra0, rb0, cb0)
    pltpu.sync_copy(cb0, gc_hbm.at[pl.ds(base + jt * CH, CH)])


def _sc_scatter_body(e_hbm, dst_hbm, zero_hbm, out_hbm, idx_v, rows0, rows1,
                     acc, rd0, rd1, sc0, sc1):
    c = lax.axis_index("c")
    s = lax.axis_index("s")
    wid = s * NC + c
    pltpu.sync_copy(zero_hbm.at[pl.ds(0, NZ)], acc.at[pl.ds(s * NZ, NZ)])

    @pl.when(s == NS - 1)
    def _():
        pltpu.sync_copy(zero_hbm.at[pl.ds(0, NTAIL)], acc.at[pl.ds(NS * NZ, NTAIL)])

    plsc.subcore_barrier()
    pltpu.sync_copy(dst_hbm.at[wid], idx_v)
    base = wid * EW

    def r_start(j, rows, sem):
        pltpu.async_copy(e_hbm.at[pl.ds(base + j * CH, CH)], rows, sem)

    def r_wait(j, rows, sem):
        pltpu.make_async_copy(e_hbm.at[pl.ds(base + j * CH, CH)], rows, sem).wait()

    def s_start(j, rows, sem):
        pltpu.async_copy(rows, acc.at[idx_v.at[j]], sem, add=True)

    def s_wait(j, rows, sem):
        pltpu.make_async_copy(rows, acc.at[idx_v.at[j]], sem).wait()

    r_start(0, rows0, rd0)
    r_start(1, rows1, rd1)

    def body(t, carry):
        j0 = 2 * t
        j1 = j0 + 1
        j2 = j0 + 2
        j3 = j0 + 3
        r_wait(j0, rows0, rd0)
        s_start(j0, rows0, sc0)
        r_wait(j1, rows1, rd1)
        s_start(j1, rows1, sc1)
        s_wait(j0, rows0, sc0)

        @pl.when(j2 < NCHUNK)
        def _():
            r_start(j2, rows0, rd0)

        s_wait(j1, rows1, sc1)

        @pl.when(j3 < NCHUNK)
        def _():
            r_start(j3, rows1, rd1)

        return carry

    lax.fori_loop(0, NCHUNK // 2, body, 0)
    jt = NCHUNK - 1
    r_wait(jt, rows0, rd0)
    pltpu.sync_copy(rows0, acc.at[idx_v.at[jt]], add=True)
    plsc.subcore_barrier()
    pltpu.sync_copy(acc.at[pl.ds(s * NZ, NZ)], out_hbm.at[c, pl.ds(s * NZ, NZ)])

    @pl.when(s == NS - 1)
    def _():
        pltpu.sync_copy(acc.at[pl.ds(NS * NZ, NTAIL)],
                        out_hbm.at[c, pl.ds(NS * NZ, NTAIL)])


@functools.cache
def _sc_kernels():
    mesh = plsc.VectorSubcoreMesh(core_axis_name="c", subcore_axis_name="s",
                                  num_cores=NC, num_subcores=NS)
    gather = pl.kernel(
        _sc_gather_body,
        out_type=jax.ShapeDtypeStruct((EH, D), jnp.float32),
        mesh=mesh,
        scratch_types=[
            pltpu.VMEM((NCHUNK, CH), jnp.int32),
            pltpu.VMEM((NCHUNK, CH), jnp.int32),
            pltpu.VMEM((CH, 2 * D), jnp.float32),
            pltpu.VMEM((CH, 2 * D), jnp.float32),
            pltpu.VMEM((CH, 2 * D), jnp.float32),
            pltpu.VMEM((CH, 2 * D), jnp.float32),
            pltpu.VMEM((CH, D), jnp.float32),
            pltpu.VMEM((CH, D), jnp.float32),
        ] + [pltpu.SemaphoreType.DMA] * 6,
    )
    scatter = pl.kernel(
        _sc_scatter_body,
        out_type=jax.ShapeDtypeStruct((NC, N, 2 * D), jnp.float32),
        mesh=mesh,
        scratch_types=[
            pltpu.VMEM((NCHUNK, CH), jnp.int32),
            pltpu.VMEM((CH, 2 * D), jnp.float32),
            pltpu.VMEM((CH, 2 * D), jnp.float32),
            pltpu.VMEM_SHARED((N, 2 * D), jnp.float32),
        ] + [pltpu.SemaphoreType.DMA] * 4,
    )
    return gather, scatter


def _sc_gather(tab, src3, dst3):
    return _sc_kernels()[0](tab, src3, dst3)


def _sc_scatter(edge_emb, dst3, zeros):
    return _sc_kernels()[1](edge_emb, dst3, zeros)


def _lin_w(p):
    return p["W"], p["b"].reshape(1, -1)


def kernel(state, dt, node_attr, edge_attr, glob_attr, params, edge_index, batch, ptr):
    src = edge_index[0]
    dst = edge_index[1]
    src3 = src.reshape(NH, NW, NCHUNK, CH)
    dst3 = dst.reshape(NH, NW, NCHUNK, CH)
    src2 = src.reshape(NH, EH, 1)
    ea_h = edge_attr.reshape(NH, EH, EDGE_A)
    batch2 = batch.reshape(N, 1)
    lo = ptr[:B].reshape(1, B)
    hi = ptr[1:B + 1].reshape(1, B)
    zeros = jnp.zeros((NZ, 2 * D), jnp.float32)

    ws1, bs1 = _lin_w(params["state_enc"]["l1"])
    ws2, bs2 = _lin_w(params["state_enc"]["l2"])
    wn1, bn1 = _lin_w(params["node_enc"]["l1"])
    wn2, bn2 = _lin_w(params["node_enc"]["l2"])
    wd1_, bd1_ = _lin_w(params["dt_enc"]["l1"])
    wd2_, bd2_ = _lin_w(params["dt_enc"]["l2"])
    wg1_, bg1_ = _lin_w(params["glob_enc"]["l1"])
    wg2_, bg2_ = _lin_w(params["glob_enc"]["l2"])
    we1, be1 = _lin_w(params["edge_enc"]["l1"])
    we2, be2 = _lin_w(params["edge_enc"]["l2"])
    wdec1, bdec1 = _lin_w(params["dec"]["l1"])
    wdec2, bdec2 = _lin_w(params["dec"]["l2"])

    layers = []
    for layer in params["gn"]:
        w1, b1 = _lin_w(layer["phi_e"]["l1"])
        w2e, b2e = _lin_w(layer["phi_e"]["l2"])
        wv1, bv1 = _lin_w(layer["phi_v"]["l1"])
        wv2, bv2 = _lin_w(layer["phi_v"]["l2"])
        wg1, bg1 = _lin_w(layer["phi_g"]["l1"])
        wg2, bg2 = _lin_w(layer["phi_g"]["l2"])
        layers.append(dict(
            wsrc=w1[:ND], wdst=w1[ND:2 * ND], w1e=w1[2 * ND:2 * ND + D],
            w1g=w1[2 * ND + D:], b1=b1, w2e=w2e, b2e=b2e,
            wvn=wv1[:ND], wva=wv1[ND:ND + D], wvg=wv1[ND + D:], bv1=bv1,
            wv2=wv2, bv2=bv2,
            wgn=wg1[:ND], wge=wg1[ND:ND + D], wgg=wg1[ND + D:], bg1=bg1,
            wg2=wg2, bg2=bg2,
        ))
    l0, l1 = layers

    glob0 = _glob_enc(dt, glob_attr, wd1_, bd1_, wd2_, bd2_, wg1_, bg1_, wg2_, bg2_)
    nemb, tab0 = _encode(state, node_attr, batch2, glob0,
                         ws1, bs1, ws2, bs2, wn1, bn1, wn2, bn2,
                         l0["wsrc"], l0["wdst"], l0["w1g"], l0["b1"])

    edges0, esums, ecnts = [], [], []
    for h in range(NH):
        gc = _sc_gather(tab0, src3[h], dst3[h])
        e, es, ec = _edge0(gc, ea_h[h], src2[h], lo, hi,
                           we1, be1, we2, be2, l0["w1e"], l0["w2e"], l0["b2e"])
        edges0.append(e)
        esums.append(es)
        ecnts.append(ec)
    agg0 = [_sc_scatter(edges0[h], dst3[h], zeros) for h in range(NH)]
    nemb1, nsum, ncnt = _node0(nemb, agg0[0], agg0[1], batch2, glob0,
                               l0["wvn"], l0["wva"], l0["wvg"], l0["bv1"],
                               l0["wv2"], l0["bv2"])
    tab1, glob1 = _globpre(nemb1, batch2, glob0, nsum, ncnt,
                           esums[0], ecnts[0], esums[1], ecnts[1],
                           l0["wgn"], l0["wge"], l0["wgg"], l0["bg1"],
                           l0["wg2"], l0["bg2"],
                           l1["wsrc"], l1["wdst"], l1["w1g"], l1["b1"])

    edges1 = []
    for h in range(NH):
        gc = _sc_gather(tab1, src3[h], dst3[h])
        edges1.append(_edge1(gc, edges0[h], l1["w1e"], l1["w2e"], l1["b2e"]))
    agg1 = [_sc_scatter(edges1[h], dst3[h], zeros) for h in range(NH)]
    out = _node1(nemb1, agg1[0], agg1[1], batch2, glob1, state,
                 l1["wvn"], l1["wva"], l1["wvg"], l1["bv1"],
                 l1["wv2"], l1["bv2"], wdec1, bdec1, wdec2, bdec2)
    return out


# speedup vs baseline: 6.2967x; 1.0228x over previous
import functools

import jax
import jax.numpy as jnp
from jax import lax
from jax.experimental import pallas as pl
from jax.experimental.pallas import tpu as pltpu
from jax.experimental.pallas import tpu_sc as plsc

N = 10000
E = 320000
B = 8
SD = 8
EDGE_A = 4
D = 64
ND = 128
GD = 128

NH = 2
EH = E // NH
NC = 2
NS = 16
NW = NC * NS
EW = EH // NW
CH = 40
NCHUNK = EW // CH
NZ = 624
NTAIL = N - NS * NZ

NBN = 1000
NBE = 4000

_SQRT_HALF = 0.7071067811865476


def _gelu(x):
    return x * 0.5 * (1.0 + lax.erf(x * _SQRT_HALF))


def _dot(a, b):
    return jnp.dot(a, b, preferred_element_type=jnp.float32)


def _mlp2(x, w1, b1, w2, b2):
    h = _gelu(_dot(x, w1[...]) + b1[...])
    return _gelu(_dot(h, w2[...]) + b2[...])


def _onehot_eq(batch_col, nb=B):
    g = lax.broadcasted_iota(jnp.int32, (1, nb), 1)
    return (batch_col == g).astype(jnp.float32)


def _onehot_range(idx_col, lo, hi):
    return ((idx_col >= lo[...]) & (idx_col < hi[...])).astype(jnp.float32)


def _fixed(shape):
    return pl.BlockSpec(shape, lambda i: (0,) * len(shape))


def _k_glob_enc(dt_ref, ga_ref, wd1, bd1, wd2, bd2, wg1, bg1, wg2, bg2, out_ref):
    h = _gelu(dt_ref[...] * wd1[...] + bd1[...])
    de = _gelu(_dot(h, wd2[...]) + bd2[...])
    ge = _mlp2(ga_ref[...], wg1, bg1, wg2, bg2)
    out_ref[...] = jnp.concatenate([de, ge], axis=1)


def _k_encode(st_ref, na_ref, b2_ref, glob_ref,
              ws1, bs1, ws2, bs2, wn1, bn1, wn2, bn2,
              wsrc, wdst, w1g, b1,
              nemb_ref, tab_ref):
    se = _mlp2(st_ref[...], ws1, bs1, ws2, bs2)
    ncoef = _mlp2(na_ref[...], wn1, bn1, wn2, bn2)
    nemb = jnp.concatenate([se, ncoef], axis=1)
    nemb_ref[...] = nemb
    gc = _dot(glob_ref[...], w1g[...])
    oh = _onehot_eq(b2_ref[...])
    a2 = _dot(nemb, wsrc[...]) + _dot(oh, gc) + b1[...]
    bd = _dot(nemb, wdst[...])
    tab_ref[...] = jnp.concatenate([a2, bd], axis=1)


def _k_edge0(gc_ref, ea_ref, src_ref, lo, hi,
             we1, be1, we2, be2, w1e, w2e, b2e,
             out_ref, esum_ref, ecnt_ref):
    i = pl.program_id(0)
    ee = _mlp2(ea_ref[...], we1, be1, we2, be2)
    pre = gc_ref[...] + _dot(ee, w1e[...])
    out = _gelu(_dot(_gelu(pre), w2e[...]) + b2e[...])
    out_ref[...] = out
    oh = _onehot_range(src_ref[...], lo, hi)
    bs = lax.dot_general(oh, out, (((0,), (0,)), ((), ())),
                         preferred_element_type=jnp.float32)
    bc = lax.dot_general(oh, jnp.ones_like(out), (((0,), (0,)), ((), ())),
                         preferred_element_type=jnp.float32)

    @pl.when(i == 0)
    def _():
        esum_ref[...] = jnp.zeros_like(esum_ref)
        ecnt_ref[...] = jnp.zeros_like(ecnt_ref)

    esum_ref[...] += bs
    ecnt_ref[...] += bc


def _k_edge1(gc_ref, ee_ref, w1e, w2e, b2e, out_ref):
    pre = gc_ref[...] + _dot(ee_ref[...], w1e[...])
    out = _gelu(_dot(_gelu(pre), w2e[...]) + b2e[...])
    out_ref[...] = out


def _node_new(nemb_ref, agg1_ref, agg2_ref, b2_ref, glob_ref,
              wvn, wva, wvg, bv1, wv2, bv2):
    agg = (agg1_ref[0, :, :D] + agg1_ref[1, :, :D]
           + agg2_ref[0, :, :D] + agg2_ref[1, :, :D])
    gv = _dot(glob_ref[...], wvg[...])
    oh = _onehot_eq(b2_ref[...])
    pre = _dot(nemb_ref[...], wvn[...]) + _dot(agg, wva[...]) + _dot(oh, gv) + bv1[...]
    return _gelu(_dot(_gelu(pre), wv2[...]) + bv2[...]), oh


def _k_node0(nemb_ref, agg1_ref, agg2_ref, b2_ref, glob_ref,
             wvn, wva, wvg, bv1, wv2, bv2,
             nnew_ref, nsum_ref, ncnt_ref):
    i = pl.program_id(0)
    nnew, oh = _node_new(nemb_ref, agg1_ref, agg2_ref, b2_ref, glob_ref,
                         wvn, wva, wvg, bv1, wv2, bv2)
    nnew_ref[...] = nnew
    bs = lax.dot_general(oh, nnew, (((0,), (0,)), ((), ())),
                         preferred_element_type=jnp.float32)
    bc = lax.dot_general(oh, jnp.ones_like(nnew), (((0,), (0,)), ((), ())),
                         preferred_element_type=jnp.float32)

    @pl.when(i == 0)
    def _():
        nsum_ref[...] = jnp.zeros_like(nsum_ref)
        ncnt_ref[...] = jnp.zeros_like(ncnt_ref)

    nsum_ref[...] += bs
    ncnt_ref[...] += bc


def _k_node1(nemb_ref, agg1_ref, agg2_ref, b2_ref, glob_ref, st_ref,
             wvn, wva, wvg, bv1, wv2, bv2, wd1, bd1, wd2, bd2,
             out_ref):
    nnew, _ = _node_new(nemb_ref, agg1_ref, agg2_ref, b2_ref, glob_ref,
                        wvn, wva, wvg, bv1, wv2, bv2)
    h = _gelu(_dot(nnew, wd1[...]) + bd1[...])
    out_ref[...] = st_ref[...] + _dot(h, wd2[...]) + bd2[...]


def _k_globpre(nemb_ref, b2_ref, glob_ref, nsum, ncnt, esum1, ecnt1, esum2, ecnt2,
               wgn, wge, wgg, bg1, wg2, bg2, wsrc, wdst, w1g, b1,
               tab_ref, glob1_ref):
    nmean = nsum[...] / jnp.maximum(ncnt[...], 1.0)
    emean = (esum1[...] + esum2[...]) / jnp.maximum(ecnt1[...] + ecnt2[...], 1.0)
    gpre = _dot(nmean, wgn[...]) + _dot(emean, wge[...]) + _dot(glob_ref[...], wgg[...]) + bg1[...]
    glob1 = _gelu(_dot(_gelu(gpre), wg2[...]) + bg2[...])
    glob1_ref[...] = glob1
    gc = _dot(glob1, w1g[...])
    oh = _onehot_eq(b2_ref[...])
    a2 = _dot(nemb_ref[...], wsrc[...]) + _dot(oh, gc) + b1[...]
    bd = _dot(nemb_ref[...], wdst[...])
    tab_ref[...] = jnp.concatenate([a2, bd], axis=1)


_TC_PARAMS = pltpu.CompilerParams(dimension_semantics=("arbitrary",))


def _glob_enc(dt, glob_attr, wd1, bd1, wd2, bd2, wg1, bg1, wg2, bg2):
    return pl.pallas_call(
        _k_glob_enc,
        out_shape=jax.ShapeDtypeStruct((B, GD), jnp.float32),
    )(dt, glob_attr, wd1, bd1, wd2, bd2, wg1, bg1, wg2, bg2)


def _encode(state, node_attr, batch2, glob0, *ws):
    grid = (N // NBN,)
    return pl.pallas_call(
        _k_encode,
        grid=grid,
        in_specs=[
            pl.BlockSpec((NBN, SD), lambda i: (i, 0)),
            pl.BlockSpec((NBN, SD), lambda i: (i, 0)),
            pl.BlockSpec((NBN, 1), lambda i: (i, 0)),
            _fixed((B, GD)),
        ] + [_fixed(w.shape) for w in ws],
        out_specs=[
            pl.BlockSpec((NBN, ND), lambda i: (i, 0)),
            pl.BlockSpec((NBN, 2 * D), lambda i: (i, 0)),
        ],
        out_shape=[
            jax.ShapeDtypeStruct((N, ND), jnp.float32),
            jax.ShapeDtypeStruct((N, 2 * D), jnp.float32),
        ],
        compiler_params=_TC_PARAMS,
    )(state, node_attr, batch2, glob0, *ws)


def _edge0(gc, edge_attr, src2, lo, hi, *ws):
    grid = (EH // NBE,)
    return pl.pallas_call(
        _k_edge0,
        grid=grid,
        in_specs=[
            pl.BlockSpec((NBE, D), lambda i: (i, 0)),
            pl.BlockSpec((NBE, 4), lambda i: (i, 0)),
            pl.BlockSpec((NBE, 1), lambda i: (i, 0)),
            _fixed((1, B)),
            _fixed((1, B)),
        ] + [_fixed(w.shape) for w in ws],
        out_specs=[
            pl.BlockSpec((NBE, D), lambda i: (i, 0)),
            _fixed((B, D)),
            _fixed((B, D)),
        ],
        out_shape=[
            jax.ShapeDtypeStruct((EH, D), jnp.float32),
            jax.ShapeDtypeStruct((B, D), jnp.float32),
            jax.ShapeDtypeStruct((B, D), jnp.float32),
        ],
        compiler_params=_TC_PARAMS,
    )(gc, edge_attr, src2, lo, hi, *ws)


def _edge1(gc, ee, *ws):
    grid = (EH // NBE,)
    return pl.pallas_call(
        _k_edge1,
        grid=grid,
        in_specs=[
            pl.BlockSpec((NBE, D), lambda i: (i, 0)),
            pl.BlockSpec((NBE, D), lambda i: (i, 0)),
        ] + [_fixed(w.shape) for w in ws],
        out_specs=pl.BlockSpec((NBE, D), lambda i: (i, 0)),
        out_shape=jax.ShapeDtypeStruct((EH, D), jnp.float32),
        compiler_params=_TC_PARAMS,
    )(gc, ee, *ws)


def _node0(nemb, agg1, agg2, batch2, glob0, *ws):
    grid = (N // NBN,)
    return pl.pallas_call(
        _k_node0,
        grid=grid,
        in_specs=[
            pl.BlockSpec((NBN, ND), lambda i: (i, 0)),
            pl.BlockSpec((NC, NBN, 2 * D), lambda i: (0, i, 0)),
            pl.BlockSpec((NC, NBN, 2 * D), lambda i: (0, i, 0)),
            pl.BlockSpec((NBN, 1), lambda i: (i, 0)),
            _fixed((B, GD)),
        ] + [_fixed(w.shape) for w in ws],
        out_specs=[
            pl.BlockSpec((NBN, ND), lambda i: (i, 0)),
            _fixed((B, ND)),
            _fixed((B, ND)),
        ],
        out_shape=[
            jax.ShapeDtypeStruct((N, ND), jnp.float32),
            jax.ShapeDtypeStruct((B, ND), jnp.float32),
            jax.ShapeDtypeStruct((B, ND), jnp.float32),
        ],
        compiler_params=_TC_PARAMS,
    )(nemb, agg1, agg2, batch2, glob0, *ws)


def _node1(nemb, agg1, agg2, batch2, glob1, state, *ws):
    grid = (N // NBN,)
    return pl.pallas_call(
        _k_node1,
        grid=grid,
        in_specs=[
            pl.BlockSpec((NBN, ND), lambda i: (i, 0)),
            pl.BlockSpec((NC, NBN, 2 * D), lambda i: (0, i, 0)),
            pl.BlockSpec((NC, NBN, 2 * D), lambda i: (0, i, 0)),
            pl.BlockSpec((NBN, 1), lambda i: (i, 0)),
            _fixed((B, GD)),
            pl.BlockSpec((NBN, SD), lambda i: (i, 0)),
        ] + [_fixed(w.shape) for w in ws],
        out_specs=pl.BlockSpec((NBN, SD), lambda i: (i, 0)),
        out_shape=jax.ShapeDtypeStruct((N, SD), jnp.float32),
        compiler_params=_TC_PARAMS,
    )(nemb, agg1, agg2, batch2, glob1, state, *ws)


def _globpre(nemb1, batch2, glob0, nsum, ncnt, esum1, ecnt1, esum2, ecnt2, *ws):
    grid = (N // NBN,)
    return pl.pallas_call(
        _k_globpre,
        grid=grid,
        in_specs=[
            pl.BlockSpec((NBN, ND), lambda i: (i, 0)),
            pl.BlockSpec((NBN, 1), lambda i: (i, 0)),
            _fixed((B, GD)),
            _fixed((B, ND)),
            _fixed((B, ND)),
            _fixed((B, D)),
            _fixed((B, D)),
            _fixed((B, D)),
            _fixed((B, D)),
        ] + [_fixed(w.shape) for w in ws],
        out_specs=[
            pl.BlockSpec((NBN, 2 * D), lambda i: (i, 0)),
            _fixed((B, GD)),
        ],
        out_shape=[
            jax.ShapeDtypeStruct((N, 2 * D), jnp.float32),
            jax.ShapeDtypeStruct((B, GD), jnp.float32),
        ],
        compiler_params=_TC_PARAMS,
    )(nemb1, batch2, glob0, nsum, ncnt, esum1, ecnt1, esum2, ecnt2, *ws)


def _sc_gather_body(tab_hbm, src_hbm, dst_hbm, gc_hbm,
                    idx_a, idx_b, ra0, rb0, ra1, rb1, cb0, cb1,
                    ga0, gb0, ga1, gb1, wa0, wb0):
    c = lax.axis_index("c")
    s = lax.axis_index("s")
    wid = s * NC + c
    base = wid * EW
    pltpu.sync_copy(src_hbm.at[wid], idx_a)
    pltpu.sync_copy(dst_hbm.at[wid], idx_b)

    def g_start(j, ra, rb, sa, sb):
        pltpu.async_copy(tab_hbm.at[idx_a.at[j]], ra, sa)
        pltpu.async_copy(tab_hbm.at[idx_b.at[j]], rb, sb)

    def g_wait(j, ra, rb, sa, sb):
        pltpu.make_async_copy(tab_hbm.at[idx_a.at[j]], ra, sa).wait()
        pltpu.make_async_copy(tab_hbm.at[idx_b.at[j]], rb, sb).wait()

    def add_halves(ra, rb, comb):
        def rbody(r, carry):
            for k in range(D // 16):
                comb[r, pl.ds(k * 16, 16)] = (ra[r, pl.ds(k * 16, 16)]
                                              + rb[r, pl.ds(D + k * 16, 16)])
            return carry
        lax.fori_loop(0, CH, rbody, 0, unroll=4)

    def w_start(j, comb, sa):
        pltpu.async_copy(comb, gc_hbm.at[pl.ds(base + j * CH, CH)], sa)

    def w_wait(j, comb, sa):
        pltpu.make_async_copy(comb, gc_hbm.at[pl.ds(base + j * CH, CH)], sa).wait()

    g_start(0, ra0, rb0, ga0, gb0)
    g_start(1, ra1, rb1, ga1, gb1)

    def body(t, carry):
        j0 = 2 * t
        j1 = j0 + 1
        j2 = j0 + 2
        j3 = j0 + 3
        g_wait(j0, ra0, rb0, ga0, gb0)

        @pl.when(t > 0)
        def _():
            w_wait(j0 - 2, cb0, wa0)

        add_halves(ra0, rb0, cb0)
        w_start(j0, cb0, wa0)

        @pl.when(j2 < NCHUNK)
        def _():
            g_start(j2, ra0, rb0, ga0, gb0)

        g_wait(j1, ra1, rb1, ga1, gb1)

        @pl.when(t > 0)
        def _():
            w_wait(j1 - 2, cb1, wb0)

        add_halves(ra1, rb1, cb1)
        w_start(j1, cb1, wb0)

        @pl.when(j3 < NCHUNK)
        def _():
            g_start(j3, ra1, rb1, ga1, gb1)

        return carry

    lax.fori_loop(0, NCHUNK // 2, body, 0)
    jt = NCHUNK - 1
    g_wait(jt, ra0, rb0, ga0, gb0)
    w_wait(jt - 2, cb0, wa0)
    w_wait(jt - 1, cb1, wb0)
    add_halves(ra0, rb0, cb0)
    pltpu.sync_copy(cb0, gc_hbm.at[pl.ds(base + jt * CH, CH)])


def _sc_scatter_body(e_hbm, dst_hbm, zero_hbm, out_hbm, idx_v, rb0, rb1,
                     u0, u1, acc, rd0, rd1, sc0, sc1):
    c = lax.axis_index("c")
    s = lax.axis_index("s")
    wid = s * NC + c
    pltpu.sync_copy(zero_hbm.at[pl.ds(0, NZ)], acc.at[pl.ds(s * NZ, NZ)])

    @pl.when(s == NS - 1)
    def _():
        pltpu.sync_copy(zero_hbm.at[pl.ds(0, NTAIL)], acc.at[pl.ds(NS * NZ, NTAIL)])

    plsc.subcore_barrier()
    pltpu.sync_copy(dst_hbm.at[wid], idx_v)
    base = wid * EW
    pltpu.sync_copy(zero_hbm.at[pl.ds(0, CH)], u0)
    pltpu.sync_copy(zero_hbm.at[pl.ds(0, CH)], u1)

    def r_start(j, rb, sem):
        pltpu.async_copy(e_hbm.at[pl.ds(base + j * CH, CH)], rb, sem)

    def r_wait(j, rb, sem):
        pltpu.make_async_copy(e_hbm.at[pl.ds(base + j * CH, CH)], rb, sem).wait()

    def fill_left(rb, u):
        def rbody(r, carry):
            for k in range(D // 16):
                u[r, pl.ds(k * 16, 16)] = rb[r, pl.ds(k * 16, 16)]
            return carry
        lax.fori_loop(0, CH, rbody, 0, unroll=4)

    def s_start(j, u, sem):
        pltpu.async_copy(u, acc.at[idx_v.at[j]], sem, add=True)

    def s_wait(j, u, sem):
        pltpu.make_async_copy(u, acc.at[idx_v.at[j]], sem).wait()

    r_start(0, rb0, rd0)
    r_start(1, rb1, rd1)

    def body(t, carry):
        j0 = 2 * t
        j1 = j0 + 1
        j2 = j0 + 2
        j3 = j0 + 3
        r_wait(j0, rb0, rd0)

        @pl.when(t > 0)
        def _():
            s_wait(j0 - 2, u0, sc0)

        fill_left(rb0, u0)

        @pl.when(j2 < NCHUNK)
        def _():
            r_start(j2, rb0, rd0)

        s_start(j0, u0, sc0)
        r_wait(j1, rb1, rd1)

        @pl.when(t > 0)
        def _():
            s_wait(j1 - 2, u1, sc1)

        fill_left(rb1, u1)

        @pl.when(j3 < NCHUNK)
        def _():
            r_start(j3, rb1, rd1)

        s_start(j1, u1, sc1)
        return carry

    lax.fori_loop(0, NCHUNK // 2, body, 0)
    jt = NCHUNK - 1
    r_wait(jt, rb0, rd0)
    s_wait(jt - 2, u0, sc0)
    s_wait(jt - 1, u1, sc1)
    fill_left(rb0, u0)
    pltpu.sync_copy(u0, acc.at[idx_v.at[jt]], add=True)
    plsc.subcore_barrier()
    pltpu.sync_copy(acc.at[pl.ds(s * NZ, NZ)], out_hbm.at[c, pl.ds(s * NZ, NZ)])

    @pl.when(s == NS - 1)
    def _():
        pltpu.sync_copy(acc.at[pl.ds(NS * NZ, NTAIL)],
                        out_hbm.at[c, pl.ds(NS * NZ, NTAIL)])


@functools.cache
def _sc_kernels():
    mesh = plsc.VectorSubcoreMesh(core_axis_name="c", subcore_axis_name="s",
                                  num_cores=NC, num_subcores=NS)
    gather = pl.kernel(
        _sc_gather_body,
        out_type=jax.ShapeDtypeStruct((EH, D), jnp.float32),
        mesh=mesh,
        scratch_types=[
            pltpu.VMEM((NCHUNK, CH), jnp.int32),
            pltpu.VMEM((NCHUNK, CH), jnp.int32),
            pltpu.VMEM((CH, 2 * D), jnp.float32),
            pltpu.VMEM((CH, 2 * D), jnp.float32),
            pltpu.VMEM((CH, 2 * D), jnp.float32),
            pltpu.VMEM((CH, 2 * D), jnp.float32),
            pltpu.VMEM((CH, D), jnp.float32),
            pltpu.VMEM((CH, D), jnp.float32),
        ] + [pltpu.SemaphoreType.DMA] * 6,
    )
    scatter = pl.kernel(
        _sc_scatter_body,
        out_type=jax.ShapeDtypeStruct((NC, N, 2 * D), jnp.float32),
        mesh=mesh,
        scratch_types=[
            pltpu.VMEM((NCHUNK, CH), jnp.int32),
            pltpu.VMEM((CH, D), jnp.float32),
            pltpu.VMEM((CH, D), jnp.float32),
            pltpu.VMEM((CH, 2 * D), jnp.float32),
            pltpu.VMEM((CH, 2 * D), jnp.float32),
            pltpu.VMEM_SHARED((N, 2 * D), jnp.float32),
        ] + [pltpu.SemaphoreType.DMA] * 4,
    )
    return gather, scatter


def _sc_gather(tab, src3, dst3):
    return _sc_kernels()[0](tab, src3, dst3)


def _sc_scatter(edge_emb, dst3, zeros):
    return _sc_kernels()[1](edge_emb, dst3, zeros)


def _lin_w(p):
    return p["W"], p["b"].reshape(1, -1)


def kernel(state, dt, node_attr, edge_attr, glob_attr, params, edge_index, batch, ptr):
    src = edge_index[0]
    dst = edge_index[1]
    src3 = src.reshape(NH, NW, NCHUNK, CH)
    dst3 = dst.reshape(NH, NW, NCHUNK, CH)
    src2 = src.reshape(NH, EH, 1)
    ea_h = edge_attr.reshape(NH, EH, EDGE_A)
    batch2 = batch.reshape(N, 1)
    lo = ptr[:B].reshape(1, B)
    hi = ptr[1:B + 1].reshape(1, B)
    zeros = jnp.zeros((NZ, 2 * D), jnp.float32)

    ws1, bs1 = _lin_w(params["state_enc"]["l1"])
    ws2, bs2 = _lin_w(params["state_enc"]["l2"])
    wn1, bn1 = _lin_w(params["node_enc"]["l1"])
    wn2, bn2 = _lin_w(params["node_enc"]["l2"])
    wd1_, bd1_ = _lin_w(params["dt_enc"]["l1"])
    wd2_, bd2_ = _lin_w(params["dt_enc"]["l2"])
    wg1_, bg1_ = _lin_w(params["glob_enc"]["l1"])
    wg2_, bg2_ = _lin_w(params["glob_enc"]["l2"])
    we1, be1 = _lin_w(params["edge_enc"]["l1"])
    we2, be2 = _lin_w(params["edge_enc"]["l2"])
    wdec1, bdec1 = _lin_w(params["dec"]["l1"])
    wdec2, bdec2 = _lin_w(params["dec"]["l2"])

    layers = []
    for layer in params["gn"]:
        w1, b1 = _lin_w(layer["phi_e"]["l1"])
        w2e, b2e = _lin_w(layer["phi_e"]["l2"])
        wv1, bv1 = _lin_w(layer["phi_v"]["l1"])
        wv2, bv2 = _lin_w(layer["phi_v"]["l2"])
        wg1, bg1 = _lin_w(layer["phi_g"]["l1"])
        wg2, bg2 = _lin_w(layer["phi_g"]["l2"])
        layers.append(dict(
            wsrc=w1[:ND], wdst=w1[ND:2 * ND], w1e=w1[2 * ND:2 * ND + D],
            w1g=w1[2 * ND + D:], b1=b1, w2e=w2e, b2e=b2e,
            wvn=wv1[:ND], wva=wv1[ND:ND + D], wvg=wv1[ND + D:], bv1=bv1,
            wv2=wv2, bv2=bv2,
            wgn=wg1[:ND], wge=wg1[ND:ND + D], wgg=wg1[ND + D:], bg1=bg1,
            wg2=wg2, bg2=bg2,
        ))
    l0, l1 = layers

    glob0 = _glob_enc(dt, glob_attr, wd1_, bd1_, wd2_, bd2_, wg1_, bg1_, wg2_, bg2_)
    nemb, tab0 = _encode(state, node_attr, batch2, glob0,
                         ws1, bs1, ws2, bs2, wn1, bn1, wn2, bn2,
                         l0["wsrc"], l0["wdst"], l0["w1g"], l0["b1"])

    edges0, esums, ecnts = [], [], []
    for h in range(NH):
        gc = _sc_gather(tab0, src3[h], dst3[h])
        e, es, ec = _edge0(gc, ea_h[h], src2[h], lo, hi,
                           we1, be1, we2, be2, l0["w1e"], l0["w2e"], l0["b2e"])
        edges0.append(e)
        esums.append(es)
        ecnts.append(ec)
    agg0 = [_sc_scatter(edges0[h], dst3[h], zeros) for h in range(NH)]
    nemb1, nsum, ncnt = _node0(nemb, agg0[0], agg0[1], batch2, glob0,
                               l0["wvn"], l0["wva"], l0["wvg"], l0["bv1"],
                               l0["wv2"], l0["bv2"])
    tab1, glob1 = _globpre(nemb1, batch2, glob0, nsum, ncnt,
                           esums[0], ecnts[0], esums[1], ecnts[1],
                           l0["wgn"], l0["wge"], l0["wgg"], l0["bg1"],
                           l0["wg2"], l0["bg2"],
                           l1["wsrc"], l1["wdst"], l1["w1g"], l1["b1"])

    edges1 = []
    for h in range(NH):
        gc = _sc_gather(tab1, src3[h], dst3[h])
        edges1.append(_edge1(gc, edges0[h], l1["w1e"], l1["w2e"], l1["b2e"]))
    agg1 = [_sc_scatter(edges1[h], dst3[h], zeros) for h in range(NH)]
    out = _node1(nemb1, agg1[0], agg1[1], batch2, glob1, state,
                 l1["wvn"], l1["wva"], l1["wvg"], l1["bv1"],
                 l1["wv2"], l1["bv2"], wdec1, bdec1, wdec2, bdec2)
    return out
